# baseline (speedup 1.0000x reference)
import functools

import jax
import jax.numpy as jnp
from jax import lax
from jax.experimental import pallas as pl
from jax.experimental.pallas import tpu as pltpu
from jax.experimental.pallas import tpu_sc as plsc

TOPK = 8
BT = 128


def _router_body(x_ref, gw_ref, logits_ref, p2_ref, wn_ref, be_ref, *, nblk):
    t, d = x_ref.shape
    e = gw_ref.shape[0]
    x = x_ref[...]
    logits = lax.dot_general(x, gw_ref[...], (((1,), (1,)), ((), ())),
                             preferred_element_type=jnp.float32)
    logits_ref[...] = logits
    m = jnp.max(logits, axis=1, keepdims=True)
    p = jnp.exp(logits - m)
    p = p / jnp.sum(p, axis=1, keepdims=True)
    lane = lax.broadcasted_iota(jnp.int32, p.shape, 1)
    work = p
    ohs, mxs = [], []
    for _ in range(TOPK):
        mx = jnp.max(work, axis=1, keepdims=True)
        eq = work == mx
        first = jnp.min(jnp.where(eq, lane, e), axis=1, keepdims=True)
        oh = lane == first
        ohs.append(oh)
        mxs.append(mx)
        work = jnp.where(oh, -1.0, work)
    mask = sum(jnp.where(oh, 1.0, 0.0) for oh in ohs)
    wsum = sum(mxs)
    ri = lax.broadcasted_iota(jnp.int32, (t, t), 0)
    ci = lax.broadcasted_iota(jnp.int32, (t, t), 1)
    tril = jnp.where(ri > ci, 1.0, 0.0)
    rank = jnp.dot(tril, mask, preferred_element_type=jnp.float32)
    counts_row = jnp.dot(jnp.ones((1, t), jnp.float32), mask,
                         preferred_element_type=jnp.float32)
    cpad_row = jnp.floor((counts_row + (BT - 1)) * (1.0 / BT)) * BT
    ui = lax.broadcasted_iota(jnp.int32, (e, e), 0)
    uj = lax.broadcasted_iota(jnp.int32, (e, e), 1)
    strict_u = jnp.where(ui < uj, 1.0, 0.0)
    offs_row = jnp.dot(cpad_row, strict_u,
                       preferred_element_type=jnp.float32)
    base = offs_row + rank
    p2_cols, wn_cols = [], []
    for k in range(TOPK):
        p2_cols.append(jnp.sum(jnp.where(ohs[k], base, 0.0), axis=1,
                               keepdims=True))
        wn_cols.append(mxs[k] / wsum)
    p2_ref[...] = jnp.concatenate(p2_cols, axis=1).astype(jnp.int32)
    wn_ref[...] = jnp.concatenate(wn_cols, axis=1)
    counts_col = lax.dot_general(mask, jnp.ones((t, 1), jnp.float32),
                                 (((0,), (0,)), ((), ())))
    cpad_col = jnp.floor((counts_col + (BT - 1)) * (1.0 / BT)) * BT
    li = lax.broadcasted_iota(jnp.int32, (e, e), 0)
    lj = lax.broadcasted_iota(jnp.int32, (e, e), 1)
    strict_l = jnp.where(li > lj, 1.0, 0.0)
    offs_col = jnp.dot(strict_l, cpad_col,
                       preferred_element_type=jnp.float32)
    bi = lax.broadcasted_iota(jnp.int32, (e, nblk), 1).astype(jnp.float32)
    ge = jnp.where(offs_col <= bi * BT, 1.0, 0.0)
    be_row = jnp.dot(jnp.ones((1, e), jnp.float32), ge,
                     preferred_element_type=jnp.float32) - 1.0
    be_ref[...] = be_row.astype(jnp.int32)


def _run_router(x, gate_w, nblk):
    t, _ = x.shape
    e = gate_w.shape[0]
    return pl.pallas_call(
        functools.partial(_router_body, nblk=nblk),
        out_shape=(
            jax.ShapeDtypeStruct((t, e), jnp.float32),
            jax.ShapeDtypeStruct((t, TOPK), jnp.int32),
            jax.ShapeDtypeStruct((t, TOPK), jnp.float32),
            jax.ShapeDtypeStruct((1, nblk), jnp.int32),
        ),
    )(x, gate_w)


def _sc_scatter_rows(x, p2m, tik, ppad):
    nr, rw = p2m.shape
    d = x.shape[1]
    rpw = nr // 32
    mesh = plsc.VectorSubcoreMesh(core_axis_name="c", subcore_axis_name="s")

    @functools.partial(
        pl.kernel,
        out_type=jax.ShapeDtypeStruct((ppad, d), x.dtype),
        mesh=mesh,
        scratch_types=[
            pltpu.VMEM((rpw, rw), jnp.int32),
            pltpu.VMEM((rpw, rw), jnp.int32),
            pltpu.VMEM((rw, d), x.dtype),
            pltpu.VMEM((rw, d), x.dtype),
            pltpu.SemaphoreType.DMA,
            pltpu.SemaphoreType.DMA,
            pltpu.SemaphoreType.DMA,
            pltpu.SemaphoreType.DMA,
        ],
    )
    def k(x_hbm, p2_hbm, tik_hbm, xs_hbm, idx_v, tik_v, rows0, rows1, gs0,
          gs1, ss0, ss1):
        w = lax.axis_index("s") * 2 + lax.axis_index("c")
        pltpu.sync_copy(p2_hbm.at[pl.ds(w * rpw, rpw)], idx_v)
        pltpu.sync_copy(tik_hbm.at[pl.ds(w * rpw, rpw)], tik_v)
        bufs = (rows0, rows1)
        gsems = (gs0, gs1)
        ssems = (ss0, ss1)
        gc = [None] * rpw
        sc = [None] * rpw
        gc[0] = pltpu.async_copy(x_hbm.at[tik_v.at[0]], bufs[0], gsems[0])
        for r in range(rpw):
            b = r % 2
            if r + 1 < rpw:
                if r >= 1:
                    sc[r - 1].wait()
                gc[r + 1] = pltpu.async_copy(x_hbm.at[tik_v.at[r + 1]],
                                             bufs[1 - b], gsems[1 - b])
            gc[r].wait()
            sc[r] = pltpu.async_copy(bufs[b], xs_hbm.at[idx_v.at[r]],
                                     ssems[b])
        if rpw >= 2:
            sc[rpw - 2].wait()
        sc[rpw - 1].wait()

    return k(x, p2m, tik)


def _sc_gather_back(ys_l, ys_r, p2m):
    nr, rw = p2m.shape
    dh = ys_l.shape[1]
    dt = ys_l.dtype
    rpw = nr // 32
    mesh = plsc.VectorSubcoreMesh(core_axis_name="c", subcore_axis_name="s")

    @functools.partial(
        pl.kernel,
        out_type=(
            jax.ShapeDtypeStruct((nr * rw, dh), dt),
            jax.ShapeDtypeStruct((nr * rw, dh), dt),
        ),
        mesh=mesh,
        scratch_types=[
            pltpu.VMEM((rpw, rw), jnp.int32),
            pltpu.VMEM((rw, dh), dt),
            pltpu.VMEM((rw, dh), dt),
            pltpu.VMEM((rw, dh), dt),
            pltpu.VMEM((rw, dh), dt),
            pltpu.SemaphoreType.DMA,
            pltpu.SemaphoreType.DMA,
            pltpu.SemaphoreType.DMA,
            pltpu.SemaphoreType.DMA,
            pltpu.SemaphoreType.DMA,
            pltpu.SemaphoreType.DMA,
            pltpu.SemaphoreType.DMA,
            pltpu.SemaphoreType.DMA,
        ],
    )
    def k(ysl_hbm, ysr_hbm, p2_hbm, ygl_hbm, ygr_hbm, idx_v, rl0, rl1, rr0,
          rr1, gl0, gl1, gr0, gr1, wl0, wl1, wr0, wr1):
        w = lax.axis_index("s") * 2 + lax.axis_index("c")
        pltpu.sync_copy(p2_hbm.at[pl.ds(w * rpw, rpw)], idx_v)
        lbuf = (rl0, rl1)
        rbuf = (rr0, rr1)
        glsem = (gl0, gl1)
        grsem = (gr0, gr1)
        wlsem = (wl0, wl1)
        wrsem = (wr0, wr1)
        glc = [None] * rpw
        grc = [None] * rpw
        wlc = [None] * rpw
        wrc = [None] * rpw
        glc[0] = pltpu.async_copy(ysl_hbm.at[idx_v.at[0]], lbuf[0], glsem[0])
        grc[0] = pltpu.async_copy(ysr_hbm.at[idx_v.at[0]], rbuf[0], grsem[0])
        for r in range(rpw):
            b = r % 2
            if r + 1 < rpw:
                if r >= 1:
                    wlc[r - 1].wait()
                    wrc[r - 1].wait()
                glc[r + 1] = pltpu.async_copy(ysl_hbm.at[idx_v.at[r + 1]],
                                              lbuf[1 - b], glsem[1 - b])
                grc[r + 1] = pltpu.async_copy(ysr_hbm.at[idx_v.at[r + 1]],
                                              rbuf[1 - b], grsem[1 - b])
            glc[r].wait()
            grc[r].wait()
            dst = pl.ds((w * rpw + r) * rw, rw)
            wlc[r] = pltpu.async_copy(lbuf[b], ygl_hbm.at[dst], wlsem[b])
            wrc[r] = pltpu.async_copy(rbuf[b], ygr_hbm.at[dst], wrsem[b])
        if rpw >= 2:
            wlc[rpw - 2].wait()
            wrc[rpw - 2].wait()
        wlc[rpw - 1].wait()
        wrc[rpw - 1].wait()

    return k(ys_l, ys_r, p2m)


def _ffn_body(be_ref, xs_ref, wg_ref, wu_ref, wd_ref, ysl_ref, ysr_ref):
    xb = xs_ref[...].astype(jnp.bfloat16)
    h = jnp.dot(xb, wg_ref[0], preferred_element_type=jnp.float32)
    u = jnp.dot(xb, wu_ref[0], preferred_element_type=jnp.float32)
    act = h * jax.nn.sigmoid(h) * u
    y = jnp.dot(act.astype(jnp.bfloat16), wd_ref[0],
                preferred_element_type=jnp.float32)
    dh = ysl_ref.shape[1]
    ysl_ref[...] = y[:, :dh].astype(ysl_ref.dtype)
    ysr_ref[...] = y[:, dh:].astype(ysr_ref.dtype)


def _run_ffn(be1d, xs, wg, wu, wd):
    nblk = be1d.shape[0]
    _, d, f = wg.shape
    ppad = xs.shape[0]
    dh = d // 2
    grid_spec = pltpu.PrefetchScalarGridSpec(
        num_scalar_prefetch=1,
        grid=(nblk,),
        in_specs=[
            pl.BlockSpec((BT, d), lambda i, be: (i, 0)),
            pl.BlockSpec((1, d, f), lambda i, be: (be[i], 0, 0)),
            pl.BlockSpec((1, d, f), lambda i, be: (be[i], 0, 0)),
            pl.BlockSpec((1, f, d), lambda i, be: (be[i], 0, 0)),
        ],
        out_specs=(
            pl.BlockSpec((BT, dh), lambda i, be: (i, 0)),
            pl.BlockSpec((BT, dh), lambda i, be: (i, 0)),
        ),
    )
    return pl.pallas_call(
        _ffn_body,
        grid_spec=grid_spec,
        out_shape=(
            jax.ShapeDtypeStruct((ppad, dh), jnp.bfloat16),
            jax.ShapeDtypeStruct((ppad, dh), jnp.bfloat16),
        ),
    )(be1d, xs, wg, wu, wd)


def _shared_body(x_ref, shg_ref, shu_ref, shd_ref, segw_ref, out_ref, *,
                 nchunk):
    j = pl.program_id(0)
    x = x_ref[...]
    xb = x.astype(jnp.bfloat16)
    g = jnp.dot(xb, shg_ref[...], preferred_element_type=jnp.float32)
    u = jnp.dot(xb, shu_ref[...], preferred_element_type=jnp.float32)
    s = jnp.dot((g * jax.nn.sigmoid(g) * u).astype(jnp.bfloat16),
                shd_ref[...], preferred_element_type=jnp.float32)

    @pl.when(j == 0)
    def _():
        out_ref[...] = s

    @pl.when(j != 0)
    def _():
        out_ref[...] += s

    @pl.when(j == nchunk - 1)
    def _():
        gate = jax.nn.sigmoid(jnp.dot(x, segw_ref[...],
                                      preferred_element_type=jnp.float32))
        out_ref[...] = gate * out_ref[...]


def _run_shared(x, sh_gate_w, sh_up_w, sh_down_w, seg_w):
    t, d = x.shape
    sf = sh_gate_w.shape[1]
    nchunk = 11 if sf % 11 == 0 else 1
    cf = sf // nchunk
    return pl.pallas_call(
        functools.partial(_shared_body, nchunk=nchunk),
        grid=(nchunk,),
        in_specs=[
            pl.BlockSpec((t, d), lambda j: (0, 0)),
            pl.BlockSpec((d, cf), lambda j: (0, j)),
            pl.BlockSpec((d, cf), lambda j: (0, j)),
            pl.BlockSpec((cf, d), lambda j: (j, 0)),
            pl.BlockSpec((d, 1), lambda j: (0, 0)),
        ],
        out_specs=pl.BlockSpec((t, d), lambda j: (0, 0)),
        out_shape=jax.ShapeDtypeStruct((t, d), jnp.float32),
    )(x, sh_gate_w, sh_up_w, sh_down_w, seg_w)


def _combine_body(ygl_ref, ygr_ref, wn_ref, sh_ref, out_ref):
    dh = ygl_ref.shape[1] // TOPK
    wn = wn_ref[...]
    acc_l = sh_ref[:, :dh]
    acc_r = sh_ref[:, dh:]
    for k in range(TOPK):
        wk = wn[:, k:k + 1]
        acc_l = acc_l + wk * ygl_ref[:, k * dh:(k + 1) * dh].astype(
            jnp.float32)
        acc_r = acc_r + wk * ygr_ref[:, k * dh:(k + 1) * dh].astype(
            jnp.float32)
    out_ref[:, :dh] = acc_l
    out_ref[:, dh:] = acc_r


def _run_combine(yg_l2, yg_r2, wn, shared):
    t, d = shared.shape
    btok = 256 if t % 256 == 0 else t
    grid = (t // btok,)
    kd = yg_l2.shape[1]
    return pl.pallas_call(
        _combine_body,
        grid=grid,
        in_specs=[
            pl.BlockSpec((btok, kd), lambda i: (i, 0)),
            pl.BlockSpec((btok, kd), lambda i: (i, 0)),
            pl.BlockSpec((btok, TOPK), lambda i: (i, 0)),
            pl.BlockSpec((btok, d), lambda i: (i, 0)),
        ],
        out_specs=pl.BlockSpec((btok, d), lambda i: (i, 0)),
        out_shape=jax.ShapeDtypeStruct((t, d), jnp.float32),
    )(yg_l2, yg_r2, wn, shared)


def kernel(hidden_states, gate_w, W_gate, W_up, W_down, sh_gate_w, sh_up_w,
           sh_down_w, shared_expert_gate_w):
    b, s, d = hidden_states.shape
    t = b * s
    e, _, f = W_gate.shape
    x = hidden_states.reshape(t, d)
    nblk = (t * TOPK) // BT + e
    ppad = nblk * BT

    logits, p2, wn, be2 = _run_router(x, gate_w, nblk)
    p2m = p2.reshape(-1, 64)
    tik = (jnp.arange(t * TOPK, dtype=jnp.int32) // TOPK).reshape(-1, 64)

    x32 = lax.bitcast_convert_type(
        hidden_states.astype(jnp.bfloat16).reshape(t, d // 2, 2), jnp.int32)
    xs32 = _sc_scatter_rows(x32, p2m, tik, ppad)
    xs = lax.bitcast_convert_type(xs32, jnp.bfloat16).reshape(ppad, d)

    ys_l, ys_r = _run_ffn(be2.reshape(nblk), xs,
                          W_gate.astype(jnp.bfloat16),
                          W_up.astype(jnp.bfloat16),
                          W_down.astype(jnp.bfloat16))

    shared = _run_shared(x, sh_gate_w.astype(jnp.bfloat16),
                         sh_up_w.astype(jnp.bfloat16),
                         sh_down_w.astype(jnp.bfloat16),
                         shared_expert_gate_w)

    dh = d // 2
    ysl32 = lax.bitcast_convert_type(ys_l.reshape(ppad, dh // 2, 2),
                                     jnp.int32)
    ysr32 = lax.bitcast_convert_type(ys_r.reshape(ppad, dh // 2, 2),
                                     jnp.int32)
    yg_l32, yg_r32 = _sc_gather_back(ysl32, ysr32, p2m)
    yg_l = lax.bitcast_convert_type(yg_l32, jnp.bfloat16).reshape(
        t, TOPK * dh)
    yg_r = lax.bitcast_convert_type(yg_r32, jnp.bfloat16).reshape(
        t, TOPK * dh)

    out = _run_combine(yg_l, yg_r, wn, shared)
    return (out.reshape(b, s, d), logits)


# speedup vs baseline: 12.2324x; 12.2324x over previous
import functools

import jax
import jax.numpy as jnp
from jax import lax
from jax.experimental import pallas as pl
from jax.experimental.pallas import tpu as pltpu
from jax.experimental.pallas import tpu_sc as plsc

TOPK = 8
BT = 128


def _router_body(x_ref, gw_ref, logits_ref, p2_ref, wn_ref, be_ref, *, nblk):
    t, d = x_ref.shape
    e = gw_ref.shape[0]
    x = x_ref[...]
    logits = lax.dot_general(x, gw_ref[...], (((1,), (1,)), ((), ())),
                             preferred_element_type=jnp.float32)
    logits_ref[...] = logits
    m = jnp.max(logits, axis=1, keepdims=True)
    p = jnp.exp(logits - m)
    p = p / jnp.sum(p, axis=1, keepdims=True)
    lane = lax.broadcasted_iota(jnp.int32, p.shape, 1)
    work = p
    ohs, mxs = [], []
    for _ in range(TOPK):
        mx = jnp.max(work, axis=1, keepdims=True)
        eq = work == mx
        first = jnp.min(jnp.where(eq, lane, e), axis=1, keepdims=True)
        oh = lane == first
        ohs.append(oh)
        mxs.append(mx)
        work = jnp.where(oh, -1.0, work)
    mask = sum(jnp.where(oh, 1.0, 0.0) for oh in ohs)
    wsum = sum(mxs)
    ri = lax.broadcasted_iota(jnp.int32, (t, t), 0)
    ci = lax.broadcasted_iota(jnp.int32, (t, t), 1)
    tril = jnp.where(ri > ci, 1.0, 0.0)
    rank = jnp.dot(tril, mask, preferred_element_type=jnp.float32)
    counts_row = jnp.dot(jnp.ones((1, t), jnp.float32), mask,
                         preferred_element_type=jnp.float32)
    cpad_row = jnp.floor((counts_row + (BT - 1)) * (1.0 / BT)) * BT
    ui = lax.broadcasted_iota(jnp.int32, (e, e), 0)
    uj = lax.broadcasted_iota(jnp.int32, (e, e), 1)
    strict_u = jnp.where(ui < uj, 1.0, 0.0)
    offs_row = jnp.dot(cpad_row, strict_u,
                       preferred_element_type=jnp.float32)
    base = offs_row + rank
    p2_cols, wn_cols = [], []
    for k in range(TOPK):
        p2_cols.append(jnp.sum(jnp.where(ohs[k], base, 0.0), axis=1,
                               keepdims=True))
        wn_cols.append(mxs[k] / wsum)
    p2_ref[...] = jnp.concatenate(p2_cols, axis=1).astype(jnp.int32)
    wn_ref[...] = jnp.concatenate(wn_cols, axis=1)
    counts_col = lax.dot_general(mask, jnp.ones((t, 1), jnp.float32),
                                 (((0,), (0,)), ((), ())))
    cpad_col = jnp.floor((counts_col + (BT - 1)) * (1.0 / BT)) * BT
    li = lax.broadcasted_iota(jnp.int32, (e, e), 0)
    lj = lax.broadcasted_iota(jnp.int32, (e, e), 1)
    strict_l = jnp.where(li > lj, 1.0, 0.0)
    offs_col = jnp.dot(strict_l, cpad_col,
                       preferred_element_type=jnp.float32)
    bi = lax.broadcasted_iota(jnp.int32, (e, nblk), 1).astype(jnp.float32)
    ge = jnp.where(offs_col <= bi * BT, 1.0, 0.0)
    be_row = jnp.dot(jnp.ones((1, e), jnp.float32), ge,
                     preferred_element_type=jnp.float32) - 1.0
    be_ref[...] = be_row.astype(jnp.int32)


def _run_router(x, gate_w, nblk):
    t, _ = x.shape
    e = gate_w.shape[0]
    return pl.pallas_call(
        functools.partial(_router_body, nblk=nblk),
        out_shape=(
            jax.ShapeDtypeStruct((t, e), jnp.float32),
            jax.ShapeDtypeStruct((t, TOPK), jnp.int32),
            jax.ShapeDtypeStruct((t, TOPK), jnp.float32),
            jax.ShapeDtypeStruct((1, nblk), jnp.int32),
        ),
    )(x, gate_w)


def _sc_scatter_rows(x, p2m, tik, ppad):
    nr, rw = p2m.shape
    d = x.shape[1]
    rpw = nr // 32
    mesh = plsc.VectorSubcoreMesh(core_axis_name="c", subcore_axis_name="s")

    @functools.partial(
        pl.kernel,
        out_type=jax.ShapeDtypeStruct((ppad, d), x.dtype),
        mesh=mesh,
        scratch_types=[
            pltpu.VMEM((rpw, rw), jnp.int32),
            pltpu.VMEM((rpw, rw), jnp.int32),
            pltpu.VMEM((rw, d), x.dtype),
            pltpu.VMEM((rw, d), x.dtype),
            pltpu.SemaphoreType.DMA,
            pltpu.SemaphoreType.DMA,
            pltpu.SemaphoreType.DMA,
            pltpu.SemaphoreType.DMA,
        ],
    )
    def k(x_hbm, p2_hbm, tik_hbm, xs_hbm, idx_v, tik_v, rows0, rows1, gs0,
          gs1, ss0, ss1):
        w = lax.axis_index("s") * 2 + lax.axis_index("c")
        pltpu.sync_copy(p2_hbm.at[pl.ds(w * rpw, rpw)], idx_v)
        pltpu.sync_copy(tik_hbm.at[pl.ds(w * rpw, rpw)], tik_v)
        bufs = (rows0, rows1)
        gsems = (gs0, gs1)
        ssems = (ss0, ss1)
        gc = [None] * rpw
        sc = [None] * rpw
        gc[0] = pltpu.async_copy(x_hbm.at[tik_v.at[0]], bufs[0], gsems[0])
        for r in range(rpw):
            b = r % 2
            if r + 1 < rpw:
                if r >= 1:
                    sc[r - 1].wait()
                gc[r + 1] = pltpu.async_copy(x_hbm.at[tik_v.at[r + 1]],
                                             bufs[1 - b], gsems[1 - b])
            gc[r].wait()
            sc[r] = pltpu.async_copy(bufs[b], xs_hbm.at[idx_v.at[r]],
                                     ssems[b])
        if rpw >= 2:
            sc[rpw - 2].wait()
        sc[rpw - 1].wait()

    return k(x, p2m, tik)


def _sc_gather_back(ys_l, ys_r, p2m):
    nr, rw = p2m.shape
    dh = ys_l.shape[1]
    dt = ys_l.dtype
    rpw = nr // 32
    mesh = plsc.VectorSubcoreMesh(core_axis_name="c", subcore_axis_name="s")

    @functools.partial(
        pl.kernel,
        out_type=(
            jax.ShapeDtypeStruct((nr * rw, dh), dt),
            jax.ShapeDtypeStruct((nr * rw, dh), dt),
        ),
        mesh=mesh,
        scratch_types=[
            pltpu.VMEM((rpw, rw), jnp.int32),
            pltpu.VMEM((rw, dh), dt),
            pltpu.VMEM((rw, dh), dt),
            pltpu.VMEM((rw, dh), dt),
            pltpu.VMEM((rw, dh), dt),
            pltpu.SemaphoreType.DMA,
            pltpu.SemaphoreType.DMA,
            pltpu.SemaphoreType.DMA,
            pltpu.SemaphoreType.DMA,
            pltpu.SemaphoreType.DMA,
            pltpu.SemaphoreType.DMA,
            pltpu.SemaphoreType.DMA,
            pltpu.SemaphoreType.DMA,
        ],
    )
    def k(ysl_hbm, ysr_hbm, p2_hbm, ygl_hbm, ygr_hbm, idx_v, rl0, rl1, rr0,
          rr1, gl0, gl1, gr0, gr1, wl0, wl1, wr0, wr1):
        w = lax.axis_index("s") * 2 + lax.axis_index("c")
        pltpu.sync_copy(p2_hbm.at[pl.ds(w * rpw, rpw)], idx_v)
        lbuf = (rl0, rl1)
        rbuf = (rr0, rr1)
        glsem = (gl0, gl1)
        grsem = (gr0, gr1)
        wlsem = (wl0, wl1)
        wrsem = (wr0, wr1)
        glc = [None] * rpw
        grc = [None] * rpw
        wlc = [None] * rpw
        wrc = [None] * rpw
        glc[0] = pltpu.async_copy(ysl_hbm.at[idx_v.at[0]], lbuf[0], glsem[0])
        grc[0] = pltpu.async_copy(ysr_hbm.at[idx_v.at[0]], rbuf[0], grsem[0])
        for r in range(rpw):
            b = r % 2
            if r + 1 < rpw:
                if r >= 1:
                    wlc[r - 1].wait()
                    wrc[r - 1].wait()
                glc[r + 1] = pltpu.async_copy(ysl_hbm.at[idx_v.at[r + 1]],
                                              lbuf[1 - b], glsem[1 - b])
                grc[r + 1] = pltpu.async_copy(ysr_hbm.at[idx_v.at[r + 1]],
                                              rbuf[1 - b], grsem[1 - b])
            glc[r].wait()
            grc[r].wait()
            dst = pl.ds((w * rpw + r) * rw, rw)
            wlc[r] = pltpu.async_copy(lbuf[b], ygl_hbm.at[dst], wlsem[b])
            wrc[r] = pltpu.async_copy(rbuf[b], ygr_hbm.at[dst], wrsem[b])
        if rpw >= 2:
            wlc[rpw - 2].wait()
            wrc[rpw - 2].wait()
        wlc[rpw - 1].wait()
        wrc[rpw - 1].wait()

    return k(ys_l, ys_r, p2m)


def _ffn_body(be_ref, xs_ref, wg_ref, wu_ref, wd_ref, ysl_ref, ysr_ref):
    xb = xs_ref[...].astype(jnp.bfloat16)
    h = jnp.dot(xb, wg_ref[0], preferred_element_type=jnp.float32)
    u = jnp.dot(xb, wu_ref[0], preferred_element_type=jnp.float32)
    act = h * jax.nn.sigmoid(h) * u
    y = jnp.dot(act.astype(jnp.bfloat16), wd_ref[0],
                preferred_element_type=jnp.float32)
    dh = ysl_ref.shape[1]
    ysl_ref[...] = y[:, :dh].astype(ysl_ref.dtype)
    ysr_ref[...] = y[:, dh:].astype(ysr_ref.dtype)


def _run_ffn(be1d, xs, wg, wu, wd):
    nblk = be1d.shape[0]
    _, d, f = wg.shape
    ppad = xs.shape[0]
    dh = d // 2
    grid_spec = pltpu.PrefetchScalarGridSpec(
        num_scalar_prefetch=1,
        grid=(nblk,),
        in_specs=[
            pl.BlockSpec((BT, d), lambda i, be: (i, 0)),
            pl.BlockSpec((1, d, f), lambda i, be: (be[i], 0, 0)),
            pl.BlockSpec((1, d, f), lambda i, be: (be[i], 0, 0)),
            pl.BlockSpec((1, f, d), lambda i, be: (be[i], 0, 0)),
        ],
        out_specs=(
            pl.BlockSpec((BT, dh), lambda i, be: (i, 0)),
            pl.BlockSpec((BT, dh), lambda i, be: (i, 0)),
        ),
    )
    return pl.pallas_call(
        _ffn_body,
        grid_spec=grid_spec,
        out_shape=(
            jax.ShapeDtypeStruct((ppad, dh), jnp.float32),
            jax.ShapeDtypeStruct((ppad, dh), jnp.float32),
        ),
    )(be1d, xs, wg, wu, wd)


def _shared_body(x_ref, shg_ref, shu_ref, shd_ref, segw_ref, out_ref, *,
                 nchunk):
    j = pl.program_id(0)
    x = x_ref[...]
    xb = x.astype(jnp.bfloat16)
    g = jnp.dot(xb, shg_ref[...], preferred_element_type=jnp.float32)
    u = jnp.dot(xb, shu_ref[...], preferred_element_type=jnp.float32)
    s = jnp.dot((g * jax.nn.sigmoid(g) * u).astype(jnp.bfloat16),
                shd_ref[...], preferred_element_type=jnp.float32)

    @pl.when(j == 0)
    def _():
        out_ref[...] = s

    @pl.when(j != 0)
    def _():
        out_ref[...] += s

    @pl.when(j == nchunk - 1)
    def _():
        gate = jax.nn.sigmoid(jnp.dot(x, segw_ref[...],
                                      preferred_element_type=jnp.float32))
        out_ref[...] = gate * out_ref[...]


def _run_shared(x, sh_gate_w, sh_up_w, sh_down_w, seg_w):
    t, d = x.shape
    sf = sh_gate_w.shape[1]
    nchunk = 11 if sf % 11 == 0 else 1
    cf = sf // nchunk
    return pl.pallas_call(
        functools.partial(_shared_body, nchunk=nchunk),
        grid=(nchunk,),
        in_specs=[
            pl.BlockSpec((t, d), lambda j: (0, 0)),
            pl.BlockSpec((d, cf), lambda j: (0, j)),
            pl.BlockSpec((d, cf), lambda j: (0, j)),
            pl.BlockSpec((cf, d), lambda j: (j, 0)),
            pl.BlockSpec((d, 1), lambda j: (0, 0)),
        ],
        out_specs=pl.BlockSpec((t, d), lambda j: (0, 0)),
        out_shape=jax.ShapeDtypeStruct((t, d), jnp.float32),
    )(x, sh_gate_w, sh_up_w, sh_down_w, seg_w)


def _combine_body(ygl_ref, ygr_ref, wn_ref, sh_ref, out_ref):
    dh = ygl_ref.shape[1] // TOPK
    wn = wn_ref[...]
    acc_l = sh_ref[:, :dh]
    acc_r = sh_ref[:, dh:]
    for k in range(TOPK):
        wk = wn[:, k:k + 1]
        acc_l = acc_l + wk * ygl_ref[:, k * dh:(k + 1) * dh].astype(
            jnp.float32)
        acc_r = acc_r + wk * ygr_ref[:, k * dh:(k + 1) * dh].astype(
            jnp.float32)
    out_ref[:, :dh] = acc_l
    out_ref[:, dh:] = acc_r


def _run_combine(yg_l2, yg_r2, wn, shared):
    t, d = shared.shape
    btok = 256 if t % 256 == 0 else t
    grid = (t // btok,)
    kd = yg_l2.shape[1]
    return pl.pallas_call(
        _combine_body,
        grid=grid,
        in_specs=[
            pl.BlockSpec((btok, kd), lambda i: (i, 0)),
            pl.BlockSpec((btok, kd), lambda i: (i, 0)),
            pl.BlockSpec((btok, TOPK), lambda i: (i, 0)),
            pl.BlockSpec((btok, d), lambda i: (i, 0)),
        ],
        out_specs=pl.BlockSpec((btok, d), lambda i: (i, 0)),
        out_shape=jax.ShapeDtypeStruct((t, d), jnp.float32),
    )(yg_l2, yg_r2, wn, shared)


def kernel(hidden_states, gate_w, W_gate, W_up, W_down, sh_gate_w, sh_up_w,
           sh_down_w, shared_expert_gate_w):
    b, s, d = hidden_states.shape
    t = b * s
    e, _, f = W_gate.shape
    x = hidden_states.reshape(t, d)
    nblk = (t * TOPK) // BT + e
    ppad = nblk * BT

    logits, p2, wn, be2 = _run_router(x, gate_w, nblk)
    p2m = p2.reshape(-1, 32)
    tik = (jnp.arange(t * TOPK, dtype=jnp.int32) // TOPK).reshape(-1, 32)

    xs = _sc_scatter_rows(x, p2m, tik, ppad)

    ys_l, ys_r = _run_ffn(be2.reshape(nblk), xs,
                          W_gate.astype(jnp.bfloat16),
                          W_up.astype(jnp.bfloat16),
                          W_down.astype(jnp.bfloat16))

    shared = _run_shared(x, sh_gate_w.astype(jnp.bfloat16),
                         sh_up_w.astype(jnp.bfloat16),
                         sh_down_w.astype(jnp.bfloat16),
                         shared_expert_gate_w)

    dh = d // 2
    yg_l, yg_r = _sc_gather_back(ys_l, ys_r, p2m)

    out = _run_combine(yg_l.reshape(t, TOPK * dh),
                       yg_r.reshape(t, TOPK * dh), wn, shared)
    return (out.reshape(b, s, d), logits)


# speedup vs baseline: 14.2806x; 1.1674x over previous
import functools

import jax
import jax.numpy as jnp
from jax import lax
from jax.experimental import pallas as pl
from jax.experimental.pallas import tpu as pltpu
from jax.experimental.pallas import tpu_sc as plsc

TOPK = 8
BT = 128


def _pack_bf16_pair(a_f32, b_f32):
    ai = lax.bitcast_convert_type(a_f32, jnp.int32)
    bi = lax.bitcast_convert_type(b_f32, jnp.int32)
    ar = ai + jnp.int32(0x7FFF) + ((ai >> 16) & 1)
    br = bi + jnp.int32(0x7FFF) + ((bi >> 16) & 1)
    return (ar & jnp.int32(-65536)) | lax.shift_right_logical(br, 16)


def _unpack_bf16_pair(p_i32):
    hi = lax.bitcast_convert_type(p_i32 & jnp.int32(-65536), jnp.float32)
    lo = lax.bitcast_convert_type(p_i32 << 16, jnp.float32)
    return hi, lo


def _router_body(x_ref, gw_ref, logits_ref, p2_ref, wn_ref, be_ref, x32_ref,
                 *, nblk):
    t, d = x_ref.shape
    e = gw_ref.shape[0]
    x = x_ref[...]
    x32_ref[...] = _pack_bf16_pair(x[:, :d // 2], x[:, d // 2:])
    logits = lax.dot_general(x, gw_ref[...], (((1,), (1,)), ((), ())),
                             preferred_element_type=jnp.float32)
    logits_ref[...] = logits
    m = jnp.max(logits, axis=1, keepdims=True)
    p = jnp.exp(logits - m)
    p = p / jnp.sum(p, axis=1, keepdims=True)
    lane = lax.broadcasted_iota(jnp.int32, p.shape, 1)
    work = p
    ohs, mxs = [], []
    for _ in range(TOPK):
        mx = jnp.max(work, axis=1, keepdims=True)
        eq = work == mx
        first = jnp.min(jnp.where(eq, lane, e), axis=1, keepdims=True)
        oh = lane == first
        ohs.append(oh)
        mxs.append(mx)
        work = jnp.where(oh, -1.0, work)
    mask = sum(jnp.where(oh, 1.0, 0.0) for oh in ohs)
    wsum = sum(mxs)
    ri = lax.broadcasted_iota(jnp.int32, (t, t), 0)
    ci = lax.broadcasted_iota(jnp.int32, (t, t), 1)
    tril = jnp.where(ri > ci, 1.0, 0.0)
    rank = jnp.dot(tril, mask, preferred_element_type=jnp.float32)
    counts_row = jnp.dot(jnp.ones((1, t), jnp.float32), mask,
                         preferred_element_type=jnp.float32)
    cpad_row = jnp.floor((counts_row + (BT - 1)) * (1.0 / BT)) * BT
    ui = lax.broadcasted_iota(jnp.int32, (e, e), 0)
    uj = lax.broadcasted_iota(jnp.int32, (e, e), 1)
    strict_u = jnp.where(ui < uj, 1.0, 0.0)
    offs_row = jnp.dot(cpad_row, strict_u,
                       preferred_element_type=jnp.float32)
    base = offs_row + rank
    p2_cols, wn_cols = [], []
    for k in range(TOPK):
        p2_cols.append(jnp.sum(jnp.where(ohs[k], base, 0.0), axis=1,
                               keepdims=True))
        wn_cols.append(mxs[k] / wsum)
    p2_ref[...] = jnp.concatenate(p2_cols, axis=1).astype(jnp.int32)
    wn_ref[...] = jnp.concatenate(wn_cols, axis=1)
    counts_col = lax.dot_general(mask, jnp.ones((t, 1), jnp.float32),
                                 (((0,), (0,)), ((), ())))
    cpad_col = jnp.floor((counts_col + (BT - 1)) * (1.0 / BT)) * BT
    li = lax.broadcasted_iota(jnp.int32, (e, e), 0)
    lj = lax.broadcasted_iota(jnp.int32, (e, e), 1)
    strict_l = jnp.where(li > lj, 1.0, 0.0)
    offs_col = jnp.dot(strict_l, cpad_col,
                       preferred_element_type=jnp.float32)
    bi = lax.broadcasted_iota(jnp.int32, (e, nblk), 1).astype(jnp.float32)
    ge = jnp.where(offs_col <= bi * BT, 1.0, 0.0)
    be_row = jnp.dot(jnp.ones((1, e), jnp.float32), ge,
                     preferred_element_type=jnp.float32) - 1.0
    be_ref[...] = be_row.astype(jnp.int32)


def _run_router(x, gate_w, nblk):
    t, d = x.shape
    e = gate_w.shape[0]
    return pl.pallas_call(
        functools.partial(_router_body, nblk=nblk),
        out_shape=(
            jax.ShapeDtypeStruct((t, e), jnp.float32),
            jax.ShapeDtypeStruct((t, TOPK), jnp.int32),
            jax.ShapeDtypeStruct((t, TOPK), jnp.float32),
            jax.ShapeDtypeStruct((1, nblk), jnp.int32),
            jax.ShapeDtypeStruct((t, d // 2), jnp.int32),
        ),
    )(x, gate_w)


def _sc_scatter_rows(x, p2m, tik, ppad):
    nr, rw = p2m.shape
    d = x.shape[1]
    rpw = nr // 32
    mesh = plsc.VectorSubcoreMesh(core_axis_name="c", subcore_axis_name="s")

    @functools.partial(
        pl.kernel,
        out_type=jax.ShapeDtypeStruct((ppad, d), x.dtype),
        mesh=mesh,
        scratch_types=[
            pltpu.VMEM((rpw, rw), jnp.int32),
            pltpu.VMEM((rpw, rw), jnp.int32),
            pltpu.VMEM((rw, d), x.dtype),
            pltpu.VMEM((rw, d), x.dtype),
            pltpu.SemaphoreType.DMA,
            pltpu.SemaphoreType.DMA,
            pltpu.SemaphoreType.DMA,
            pltpu.SemaphoreType.DMA,
        ],
    )
    def k(x_hbm, p2_hbm, tik_hbm, xs_hbm, idx_v, tik_v, rows0, rows1, gs0,
          gs1, ss0, ss1):
        w = lax.axis_index("s") * 2 + lax.axis_index("c")
        pltpu.sync_copy(p2_hbm.at[pl.ds(w * rpw, rpw)], idx_v)
        pltpu.sync_copy(tik_hbm.at[pl.ds(w * rpw, rpw)], tik_v)
        bufs = (rows0, rows1)
        gsems = (gs0, gs1)
        ssems = (ss0, ss1)
        gc = [None] * rpw
        sc = [None] * rpw
        gc[0] = pltpu.async_copy(x_hbm.at[tik_v.at[0]], bufs[0], gsems[0])
        for r in range(rpw):
            b = r % 2
            if r + 1 < rpw:
                if r >= 1:
                    sc[r - 1].wait()
                gc[r + 1] = pltpu.async_copy(x_hbm.at[tik_v.at[r + 1]],
                                             bufs[1 - b], gsems[1 - b])
            gc[r].wait()
            sc[r] = pltpu.async_copy(bufs[b], xs_hbm.at[idx_v.at[r]],
                                     ssems[b])
        if rpw >= 2:
            sc[rpw - 2].wait()
        sc[rpw - 1].wait()

    return k(x, p2m, tik)


def _sc_gather_back(ys_l, ys_r, p2m):
    nr, rw = p2m.shape
    dh = ys_l.shape[1]
    dt = ys_l.dtype
    rpw = nr // 32
    mesh = plsc.VectorSubcoreMesh(core_axis_name="c", subcore_axis_name="s")

    @functools.partial(
        pl.kernel,
        out_type=(
            jax.ShapeDtypeStruct((nr * rw, dh), dt),
            jax.ShapeDtypeStruct((nr * rw, dh), dt),
        ),
        mesh=mesh,
        scratch_types=[
            pltpu.VMEM((rpw, rw), jnp.int32),
            pltpu.VMEM((rw, dh), dt),
            pltpu.VMEM((rw, dh), dt),
            pltpu.VMEM((rw, dh), dt),
            pltpu.VMEM((rw, dh), dt),
            pltpu.SemaphoreType.DMA,
            pltpu.SemaphoreType.DMA,
            pltpu.SemaphoreType.DMA,
            pltpu.SemaphoreType.DMA,
            pltpu.SemaphoreType.DMA,
            pltpu.SemaphoreType.DMA,
            pltpu.SemaphoreType.DMA,
            pltpu.SemaphoreType.DMA,
        ],
    )
    def k(ysl_hbm, ysr_hbm, p2_hbm, ygl_hbm, ygr_hbm, idx_v, rl0, rl1, rr0,
          rr1, gl0, gl1, gr0, gr1, wl0, wl1, wr0, wr1):
        w = lax.axis_index("s") * 2 + lax.axis_index("c")
        pltpu.sync_copy(p2_hbm.at[pl.ds(w * rpw, rpw)], idx_v)
        lbuf = (rl0, rl1)
        rbuf = (rr0, rr1)
        glsem = (gl0, gl1)
        grsem = (gr0, gr1)
        wlsem = (wl0, wl1)
        wrsem = (wr0, wr1)
        glc = [None] * rpw
        grc = [None] * rpw
        wlc = [None] * rpw
        wrc = [None] * rpw
        glc[0] = pltpu.async_copy(ysl_hbm.at[idx_v.at[0]], lbuf[0], glsem[0])
        grc[0] = pltpu.async_copy(ysr_hbm.at[idx_v.at[0]], rbuf[0], grsem[0])
        for r in range(rpw):
            b = r % 2
            if r + 1 < rpw:
                if r >= 1:
                    wlc[r - 1].wait()
                    wrc[r - 1].wait()
                glc[r + 1] = pltpu.async_copy(ysl_hbm.at[idx_v.at[r + 1]],
                                              lbuf[1 - b], glsem[1 - b])
                grc[r + 1] = pltpu.async_copy(ysr_hbm.at[idx_v.at[r + 1]],
                                              rbuf[1 - b], grsem[1 - b])
            glc[r].wait()
            grc[r].wait()
            dst = pl.ds((w * rpw + r) * rw, rw)
            wlc[r] = pltpu.async_copy(lbuf[b], ygl_hbm.at[dst], wlsem[b])
            wrc[r] = pltpu.async_copy(rbuf[b], ygr_hbm.at[dst], wrsem[b])
        if rpw >= 2:
            wlc[rpw - 2].wait()
            wrc[rpw - 2].wait()
        wlc[rpw - 1].wait()
        wrc[rpw - 1].wait()

    return k(ys_l, ys_r, p2m)


def _ffn_body(be_ref, xs_ref, wg_ref, wu_ref, wd_ref, ysl_ref, ysr_ref):
    xhi, xlo = _unpack_bf16_pair(xs_ref[...])
    xb = jnp.concatenate([xhi, xlo], axis=1).astype(jnp.bfloat16)
    h = jnp.dot(xb, wg_ref[0], preferred_element_type=jnp.float32)
    u = jnp.dot(xb, wu_ref[0], preferred_element_type=jnp.float32)
    act = h * jax.nn.sigmoid(h) * u
    y = jnp.dot(act.astype(jnp.bfloat16), wd_ref[0],
                preferred_element_type=jnp.float32)
    d = y.shape[1]
    dh = d // 2
    dq = d // 4
    yl = y[:, :dh]
    yr = y[:, dh:]
    ysl_ref[...] = _pack_bf16_pair(yl[:, :dq], yl[:, dq:])
    ysr_ref[...] = _pack_bf16_pair(yr[:, :dq], yr[:, dq:])


def _run_ffn(be1d, xs32, wg, wu, wd):
    nblk = be1d.shape[0]
    _, d, f = wg.shape
    ppad = xs32.shape[0]
    dq = d // 4
    grid_spec = pltpu.PrefetchScalarGridSpec(
        num_scalar_prefetch=1,
        grid=(nblk,),
        in_specs=[
            pl.BlockSpec((BT, d // 2), lambda i, be: (i, 0)),
            pl.BlockSpec((1, d, f), lambda i, be: (be[i], 0, 0)),
            pl.BlockSpec((1, d, f), lambda i, be: (be[i], 0, 0)),
            pl.BlockSpec((1, f, d), lambda i, be: (be[i], 0, 0)),
        ],
        out_specs=(
            pl.BlockSpec((BT, dq), lambda i, be: (i, 0)),
            pl.BlockSpec((BT, dq), lambda i, be: (i, 0)),
        ),
    )
    return pl.pallas_call(
        _ffn_body,
        grid_spec=grid_spec,
        out_shape=(
            jax.ShapeDtypeStruct((ppad, dq), jnp.int32),
            jax.ShapeDtypeStruct((ppad, dq), jnp.int32),
        ),
    )(be1d, xs32, wg, wu, wd)


def _shared_body(x_ref, shg_ref, shu_ref, shd_ref, segw_ref, out_ref, *,
                 nchunk):
    j = pl.program_id(0)
    x = x_ref[...]
    xb = x.astype(jnp.bfloat16)
    g = jnp.dot(xb, shg_ref[...], preferred_element_type=jnp.float32)
    u = jnp.dot(xb, shu_ref[...], preferred_element_type=jnp.float32)
    s = jnp.dot((g * jax.nn.sigmoid(g) * u).astype(jnp.bfloat16),
                shd_ref[...], preferred_element_type=jnp.float32)

    @pl.when(j == 0)
    def _():
        out_ref[...] = s

    @pl.when(j != 0)
    def _():
        out_ref[...] += s

    @pl.when(j == nchunk - 1)
    def _():
        gate = jax.nn.sigmoid(jnp.dot(x, segw_ref[...],
                                      preferred_element_type=jnp.float32))
        out_ref[...] = gate * out_ref[...]


def _run_shared(x, sh_gate_w, sh_up_w, sh_down_w, seg_w):
    t, d = x.shape
    sf = sh_gate_w.shape[1]
    nchunk = 11 if sf % 11 == 0 else 1
    cf = sf // nchunk
    return pl.pallas_call(
        functools.partial(_shared_body, nchunk=nchunk),
        grid=(nchunk,),
        in_specs=[
            pl.BlockSpec((t, d), lambda j: (0, 0)),
            pl.BlockSpec((d, cf), lambda j: (0, j)),
            pl.BlockSpec((d, cf), lambda j: (0, j)),
            pl.BlockSpec((cf, d), lambda j: (j, 0)),
            pl.BlockSpec((d, 1), lambda j: (0, 0)),
        ],
        out_specs=pl.BlockSpec((t, d), lambda j: (0, 0)),
        out_shape=jax.ShapeDtypeStruct((t, d), jnp.float32),
    )(x, sh_gate_w, sh_up_w, sh_down_w, seg_w)


def _combine_body(ygl_ref, ygr_ref, wn_ref, sh_ref, out_ref):
    dq = ygl_ref.shape[1] // TOPK
    dh = 2 * dq
    wn = wn_ref[...]
    acc = [sh_ref[:, i * dq:(i + 1) * dq] for i in range(4)]
    for k in range(TOPK):
        wk = wn[:, k:k + 1]
        lhi, llo = _unpack_bf16_pair(ygl_ref[:, k * dq:(k + 1) * dq])
        rhi, rlo = _unpack_bf16_pair(ygr_ref[:, k * dq:(k + 1) * dq])
        acc[0] = acc[0] + wk * lhi
        acc[1] = acc[1] + wk * llo
        acc[2] = acc[2] + wk * rhi
        acc[3] = acc[3] + wk * rlo
    for i in range(4):
        out_ref[:, i * dq:(i + 1) * dq] = acc[i]


def _run_combine(yg_l2, yg_r2, wn, shared):
    t, d = shared.shape
    btok = 256 if t % 256 == 0 else t
    grid = (t // btok,)
    kd = yg_l2.shape[1]
    return pl.pallas_call(
        _combine_body,
        grid=grid,
        in_specs=[
            pl.BlockSpec((btok, kd), lambda i: (i, 0)),
            pl.BlockSpec((btok, kd), lambda i: (i, 0)),
            pl.BlockSpec((btok, TOPK), lambda i: (i, 0)),
            pl.BlockSpec((btok, d), lambda i: (i, 0)),
        ],
        out_specs=pl.BlockSpec((btok, d), lambda i: (i, 0)),
        out_shape=jax.ShapeDtypeStruct((t, d), jnp.float32),
    )(yg_l2, yg_r2, wn, shared)


def kernel(hidden_states, gate_w, W_gate, W_up, W_down, sh_gate_w, sh_up_w,
           sh_down_w, shared_expert_gate_w):
    b, s, d = hidden_states.shape
    t = b * s
    e, _, f = W_gate.shape
    x = hidden_states.reshape(t, d)
    nblk = (t * TOPK) // BT + e
    ppad = nblk * BT

    logits, p2, wn, be2, x32 = _run_router(x, gate_w, nblk)
    p2m = p2.reshape(-1, 64)
    tik = (jnp.arange(t * TOPK, dtype=jnp.int32) // TOPK).reshape(-1, 64)

    xs32 = _sc_scatter_rows(x32, p2m, tik, ppad)

    ys_l, ys_r = _run_ffn(be2.reshape(nblk), xs32,
                          W_gate.astype(jnp.bfloat16),
                          W_up.astype(jnp.bfloat16),
                          W_down.astype(jnp.bfloat16))

    shared = _run_shared(x, sh_gate_w.astype(jnp.bfloat16),
                         sh_up_w.astype(jnp.bfloat16),
                         sh_down_w.astype(jnp.bfloat16),
                         shared_expert_gate_w)

    dq = d // 4
    yg_l, yg_r = _sc_gather_back(ys_l, ys_r, p2m)

    out = _run_combine(yg_l.reshape(t, TOPK * dq),
                       yg_r.reshape(t, TOPK * dq), wn, shared)
    return (out.reshape(b, s, d), logits)


# speedup vs baseline: 15.3443x; 1.0745x over previous
import functools

import jax
import jax.numpy as jnp
from jax import lax
from jax.experimental import pallas as pl
from jax.experimental.pallas import tpu as pltpu
from jax.experimental.pallas import tpu_sc as plsc

TOPK = 8
BT = 256


def _pack_bf16_pair(a_f32, b_f32):
    ai = lax.bitcast_convert_type(a_f32, jnp.int32)
    bi = lax.bitcast_convert_type(b_f32, jnp.int32)
    ar = ai + jnp.int32(0x7FFF) + ((ai >> 16) & 1)
    br = bi + jnp.int32(0x7FFF) + ((bi >> 16) & 1)
    return (ar & jnp.int32(-65536)) | lax.shift_right_logical(br, 16)


def _unpack_bf16_pair(p_i32):
    hi = lax.bitcast_convert_type(p_i32 & jnp.int32(-65536), jnp.float32)
    lo = lax.bitcast_convert_type(p_i32 << 16, jnp.float32)
    return hi, lo


def _router_body(x_ref, gw_ref, logits_ref, p2_ref, wn_ref, be_ref, x32_ref,
                 *, nblk):
    t, d = x_ref.shape
    e = gw_ref.shape[0]
    x = x_ref[...]
    x32_ref[...] = _pack_bf16_pair(x[:, :d // 2], x[:, d // 2:])
    logits = lax.dot_general(x, gw_ref[...], (((1,), (1,)), ((), ())),
                             preferred_element_type=jnp.float32)
    logits_ref[...] = logits
    m = jnp.max(logits, axis=1, keepdims=True)
    p = jnp.exp(logits - m)
    p = p / jnp.sum(p, axis=1, keepdims=True)
    lane = lax.broadcasted_iota(jnp.int32, p.shape, 1)
    work = p
    ohs, mxs = [], []
    for _ in range(TOPK):
        mx = jnp.max(work, axis=1, keepdims=True)
        eq = work == mx
        first = jnp.min(jnp.where(eq, lane, e), axis=1, keepdims=True)
        oh = lane == first
        ohs.append(oh)
        mxs.append(mx)
        work = jnp.where(oh, -1.0, work)
    mask = sum(jnp.where(oh, 1.0, 0.0) for oh in ohs)
    wsum = sum(mxs)
    ri = lax.broadcasted_iota(jnp.int32, (t, t), 0)
    ci = lax.broadcasted_iota(jnp.int32, (t, t), 1)
    tril = jnp.where(ri > ci, 1.0, 0.0)
    rank = jnp.dot(tril, mask, preferred_element_type=jnp.float32)
    counts_row = jnp.dot(jnp.ones((1, t), jnp.float32), mask,
                         preferred_element_type=jnp.float32)
    cpad_row = jnp.floor((counts_row + (BT - 1)) * (1.0 / BT)) * BT
    ui = lax.broadcasted_iota(jnp.int32, (e, e), 0)
    uj = lax.broadcasted_iota(jnp.int32, (e, e), 1)
    strict_u = jnp.where(ui < uj, 1.0, 0.0)
    offs_row = jnp.dot(cpad_row, strict_u,
                       preferred_element_type=jnp.float32)
    base = offs_row + rank
    p2_cols, wn_cols = [], []
    for k in range(TOPK):
        p2_cols.append(jnp.sum(jnp.where(ohs[k], base, 0.0), axis=1,
                               keepdims=True))
        wn_cols.append(mxs[k] / wsum)
    p2_ref[...] = jnp.concatenate(p2_cols, axis=1).astype(jnp.int32)
    wn_ref[...] = jnp.concatenate(wn_cols, axis=1)
    counts_col = lax.dot_general(mask, jnp.ones((t, 1), jnp.float32),
                                 (((0,), (0,)), ((), ())))
    cpad_col = jnp.floor((counts_col + (BT - 1)) * (1.0 / BT)) * BT
    li = lax.broadcasted_iota(jnp.int32, (e, e), 0)
    lj = lax.broadcasted_iota(jnp.int32, (e, e), 1)
    strict_l = jnp.where(li > lj, 1.0, 0.0)
    offs_col = jnp.dot(strict_l, cpad_col,
                       preferred_element_type=jnp.float32)
    bi = lax.broadcasted_iota(jnp.int32, (e, nblk), 1).astype(jnp.float32)
    ge = jnp.where(offs_col <= bi * BT, 1.0, 0.0)
    be_row = jnp.dot(jnp.ones((1, e), jnp.float32), ge,
                     preferred_element_type=jnp.float32) - 1.0
    be_ref[...] = be_row.astype(jnp.int32)


def _run_router(x, gate_w, nblk):
    t, d = x.shape
    e = gate_w.shape[0]
    return pl.pallas_call(
        functools.partial(_router_body, nblk=nblk),
        out_shape=(
            jax.ShapeDtypeStruct((t, e), jnp.float32),
            jax.ShapeDtypeStruct((t, TOPK), jnp.int32),
            jax.ShapeDtypeStruct((t, TOPK), jnp.float32),
            jax.ShapeDtypeStruct((1, nblk), jnp.int32),
            jax.ShapeDtypeStruct((t, d // 2), jnp.int32),
        ),
    )(x, gate_w)


def _sc_scatter_rows(x, p2m, tik, ppad):
    nr, rw = p2m.shape
    d = x.shape[1]
    rpw = nr // 32
    mesh = plsc.VectorSubcoreMesh(core_axis_name="c", subcore_axis_name="s")

    @functools.partial(
        pl.kernel,
        out_type=jax.ShapeDtypeStruct((ppad, d), x.dtype),
        mesh=mesh,
        scratch_types=[
            pltpu.VMEM((rpw, rw), jnp.int32),
            pltpu.VMEM((rpw, rw), jnp.int32),
            pltpu.VMEM((rw, d), x.dtype),
            pltpu.VMEM((rw, d), x.dtype),
            pltpu.SemaphoreType.DMA,
            pltpu.SemaphoreType.DMA,
            pltpu.SemaphoreType.DMA,
            pltpu.SemaphoreType.DMA,
        ],
    )
    def k(x_hbm, p2_hbm, tik_hbm, xs_hbm, idx_v, tik_v, rows0, rows1, gs0,
          gs1, ss0, ss1):
        w = lax.axis_index("s") * 2 + lax.axis_index("c")
        pltpu.sync_copy(p2_hbm.at[pl.ds(w * rpw, rpw)], idx_v)
        pltpu.sync_copy(tik_hbm.at[pl.ds(w * rpw, rpw)], tik_v)
        bufs = (rows0, rows1)
        gsems = (gs0, gs1)
        ssems = (ss0, ss1)
        gc = [None] * rpw
        sc = [None] * rpw
        gc[0] = pltpu.async_copy(x_hbm.at[tik_v.at[0]], bufs[0], gsems[0])
        for r in range(rpw):
            b = r % 2
            if r + 1 < rpw:
                if r >= 1:
                    sc[r - 1].wait()
                gc[r + 1] = pltpu.async_copy(x_hbm.at[tik_v.at[r + 1]],
                                             bufs[1 - b], gsems[1 - b])
            gc[r].wait()
            sc[r] = pltpu.async_copy(bufs[b], xs_hbm.at[idx_v.at[r]],
                                     ssems[b])
        if rpw >= 2:
            sc[rpw - 2].wait()
        sc[rpw - 1].wait()

    return k(x, p2m, tik)


def _sc_gather_back(ys_l, ys_r, p2m):
    nr, rw = p2m.shape
    dh = ys_l.shape[1]
    dt = ys_l.dtype
    rpw = nr // 32
    mesh = plsc.VectorSubcoreMesh(core_axis_name="c", subcore_axis_name="s")

    @functools.partial(
        pl.kernel,
        out_type=(
            jax.ShapeDtypeStruct((nr * rw, dh), dt),
            jax.ShapeDtypeStruct((nr * rw, dh), dt),
        ),
        mesh=mesh,
        scratch_types=[
            pltpu.VMEM((rpw, rw), jnp.int32),
            pltpu.VMEM((rw, dh), dt),
            pltpu.VMEM((rw, dh), dt),
            pltpu.VMEM((rw, dh), dt),
            pltpu.VMEM((rw, dh), dt),
            pltpu.SemaphoreType.DMA,
            pltpu.SemaphoreType.DMA,
            pltpu.SemaphoreType.DMA,
            pltpu.SemaphoreType.DMA,
            pltpu.SemaphoreType.DMA,
            pltpu.SemaphoreType.DMA,
            pltpu.SemaphoreType.DMA,
            pltpu.SemaphoreType.DMA,
        ],
    )
    def k(ysl_hbm, ysr_hbm, p2_hbm, ygl_hbm, ygr_hbm, idx_v, rl0, rl1, rr0,
          rr1, gl0, gl1, gr0, gr1, wl0, wl1, wr0, wr1):
        w = lax.axis_index("s") * 2 + lax.axis_index("c")
        pltpu.sync_copy(p2_hbm.at[pl.ds(w * rpw, rpw)], idx_v)
        lbuf = (rl0, rl1)
        rbuf = (rr0, rr1)
        glsem = (gl0, gl1)
        grsem = (gr0, gr1)
        wlsem = (wl0, wl1)
        wrsem = (wr0, wr1)
        glc = [None] * rpw
        grc = [None] * rpw
        wlc = [None] * rpw
        wrc = [None] * rpw
        glc[0] = pltpu.async_copy(ysl_hbm.at[idx_v.at[0]], lbuf[0], glsem[0])
        grc[0] = pltpu.async_copy(ysr_hbm.at[idx_v.at[0]], rbuf[0], grsem[0])
        for r in range(rpw):
            b = r % 2
            if r + 1 < rpw:
                if r >= 1:
                    wlc[r - 1].wait()
                    wrc[r - 1].wait()
                glc[r + 1] = pltpu.async_copy(ysl_hbm.at[idx_v.at[r + 1]],
                                              lbuf[1 - b], glsem[1 - b])
                grc[r + 1] = pltpu.async_copy(ysr_hbm.at[idx_v.at[r + 1]],
                                              rbuf[1 - b], grsem[1 - b])
            glc[r].wait()
            grc[r].wait()
            dst = pl.ds((w * rpw + r) * rw, rw)
            wlc[r] = pltpu.async_copy(lbuf[b], ygl_hbm.at[dst], wlsem[b])
            wrc[r] = pltpu.async_copy(rbuf[b], ygr_hbm.at[dst], wrsem[b])
        if rpw >= 2:
            wlc[rpw - 2].wait()
            wrc[rpw - 2].wait()
        wlc[rpw - 1].wait()
        wrc[rpw - 1].wait()

    return k(ys_l, ys_r, p2m)


def _ffn_body(be_ref, xs_ref, wg_ref, wu_ref, wd_ref, ysl_ref, ysr_ref):
    xhi, xlo = _unpack_bf16_pair(xs_ref[...])
    xb = jnp.concatenate([xhi, xlo], axis=1).astype(jnp.bfloat16)
    h = jnp.dot(xb, wg_ref[0], preferred_element_type=jnp.float32)
    u = jnp.dot(xb, wu_ref[0], preferred_element_type=jnp.float32)
    act = h * jax.nn.sigmoid(h) * u
    y = jnp.dot(act.astype(jnp.bfloat16), wd_ref[0],
                preferred_element_type=jnp.float32)
    d = y.shape[1]
    dh = d // 2
    dq = d // 4
    yl = y[:, :dh]
    yr = y[:, dh:]
    ysl_ref[...] = _pack_bf16_pair(yl[:, :dq], yl[:, dq:])
    ysr_ref[...] = _pack_bf16_pair(yr[:, :dq], yr[:, dq:])


def _run_ffn(be1d, xs32, wg, wu, wd):
    nblk = be1d.shape[0]
    _, d, f = wg.shape
    ppad = xs32.shape[0]
    dq = d // 4
    grid_spec = pltpu.PrefetchScalarGridSpec(
        num_scalar_prefetch=1,
        grid=(nblk,),
        in_specs=[
            pl.BlockSpec((BT, d // 2), lambda i, be: (i, 0)),
            pl.BlockSpec((1, d, f), lambda i, be: (be[i], 0, 0)),
            pl.BlockSpec((1, d, f), lambda i, be: (be[i], 0, 0)),
            pl.BlockSpec((1, f, d), lambda i, be: (be[i], 0, 0)),
        ],
        out_specs=(
            pl.BlockSpec((BT, dq), lambda i, be: (i, 0)),
            pl.BlockSpec((BT, dq), lambda i, be: (i, 0)),
        ),
    )
    return pl.pallas_call(
        _ffn_body,
        grid_spec=grid_spec,
        out_shape=(
            jax.ShapeDtypeStruct((ppad, dq), jnp.int32),
            jax.ShapeDtypeStruct((ppad, dq), jnp.int32),
        ),
    )(be1d, xs32, wg, wu, wd)


def _shared_body(x_ref, shg_ref, shu_ref, shd_ref, segw_ref, out_ref, *,
                 nchunk):
    j = pl.program_id(1)
    x = x_ref[...]
    xb = x.astype(jnp.bfloat16)
    g = jnp.dot(xb, shg_ref[...], preferred_element_type=jnp.float32)
    u = jnp.dot(xb, shu_ref[...], preferred_element_type=jnp.float32)
    s = jnp.dot((g * jax.nn.sigmoid(g) * u).astype(jnp.bfloat16),
                shd_ref[...], preferred_element_type=jnp.float32)

    @pl.when(j == 0)
    def _():
        out_ref[...] = s

    @pl.when(j != 0)
    def _():
        out_ref[...] += s

    @pl.when(j == nchunk - 1)
    def _():
        gate = jax.nn.sigmoid(jnp.dot(x, segw_ref[...],
                                      preferred_element_type=jnp.float32))
        out_ref[...] = gate * out_ref[...]


def _run_shared(x, sh_gate_w, sh_up_w, sh_down_w, seg_w):
    t, d = x.shape
    sf = sh_gate_w.shape[1]
    nchunk = 2 if sf % 2 == 0 else 1
    cf = sf // nchunk
    bt = 512 if t % 512 == 0 else t
    return pl.pallas_call(
        functools.partial(_shared_body, nchunk=nchunk),
        grid=(t // bt, nchunk),
        in_specs=[
            pl.BlockSpec((bt, d), lambda i, j: (i, 0)),
            pl.BlockSpec((d, cf), lambda i, j: (0, j)),
            pl.BlockSpec((d, cf), lambda i, j: (0, j)),
            pl.BlockSpec((cf, d), lambda i, j: (j, 0)),
            pl.BlockSpec((d, 1), lambda i, j: (0, 0)),
        ],
        out_specs=pl.BlockSpec((bt, d), lambda i, j: (i, 0)),
        out_shape=jax.ShapeDtypeStruct((t, d), jnp.float32),
    )(x, sh_gate_w, sh_up_w, sh_down_w, seg_w)


def _combine_body(ygl_ref, ygr_ref, wn_ref, sh_ref, out_ref):
    dq = ygl_ref.shape[1] // TOPK
    dh = 2 * dq
    wn = wn_ref[...]
    acc = [sh_ref[:, i * dq:(i + 1) * dq] for i in range(4)]
    for k in range(TOPK):
        wk = wn[:, k:k + 1]
        lhi, llo = _unpack_bf16_pair(ygl_ref[:, k * dq:(k + 1) * dq])
        rhi, rlo = _unpack_bf16_pair(ygr_ref[:, k * dq:(k + 1) * dq])
        acc[0] = acc[0] + wk * lhi
        acc[1] = acc[1] + wk * llo
        acc[2] = acc[2] + wk * rhi
        acc[3] = acc[3] + wk * rlo
    for i in range(4):
        out_ref[:, i * dq:(i + 1) * dq] = acc[i]


def _run_combine(yg_l2, yg_r2, wn, shared):
    t, d = shared.shape
    btok = 256 if t % 256 == 0 else t
    grid = (t // btok,)
    kd = yg_l2.shape[1]
    return pl.pallas_call(
        _combine_body,
        grid=grid,
        in_specs=[
            pl.BlockSpec((btok, kd), lambda i: (i, 0)),
            pl.BlockSpec((btok, kd), lambda i: (i, 0)),
            pl.BlockSpec((btok, TOPK), lambda i: (i, 0)),
            pl.BlockSpec((btok, d), lambda i: (i, 0)),
        ],
        out_specs=pl.BlockSpec((btok, d), lambda i: (i, 0)),
        out_shape=jax.ShapeDtypeStruct((t, d), jnp.float32),
    )(yg_l2, yg_r2, wn, shared)


def kernel(hidden_states, gate_w, W_gate, W_up, W_down, sh_gate_w, sh_up_w,
           sh_down_w, shared_expert_gate_w):
    b, s, d = hidden_states.shape
    t = b * s
    e, _, f = W_gate.shape
    x = hidden_states.reshape(t, d)
    nblk = (t * TOPK) // BT + e
    ppad = nblk * BT

    logits, p2, wn, be2, x32 = _run_router(x, gate_w, nblk)
    p2m = p2.reshape(-1, 64)
    tik = (jnp.arange(t * TOPK, dtype=jnp.int32) // TOPK).reshape(-1, 64)

    xs32 = _sc_scatter_rows(x32, p2m, tik, ppad)

    ys_l, ys_r = _run_ffn(be2.reshape(nblk), xs32,
                          W_gate.astype(jnp.bfloat16),
                          W_up.astype(jnp.bfloat16),
                          W_down.astype(jnp.bfloat16))

    shared = _run_shared(x, sh_gate_w.astype(jnp.bfloat16),
                         sh_up_w.astype(jnp.bfloat16),
                         sh_down_w.astype(jnp.bfloat16),
                         shared_expert_gate_w)

    dq = d // 4
    yg_l, yg_r = _sc_gather_back(ys_l, ys_r, p2m)

    out = _run_combine(yg_l.reshape(t, TOPK * dq),
                       yg_r.reshape(t, TOPK * dq), wn, shared)
    return (out.reshape(b, s, d), logits)


# speedup vs baseline: 15.4255x; 1.0053x over previous
import functools

import jax
import jax.numpy as jnp
from jax import lax
from jax.experimental import pallas as pl
from jax.experimental.pallas import tpu as pltpu
from jax.experimental.pallas import tpu_sc as plsc

TOPK = 8
BT = 256


def _pack_bf16_pair(a_f32, b_f32):
    ai = lax.bitcast_convert_type(a_f32, jnp.int32)
    bi = lax.bitcast_convert_type(b_f32, jnp.int32)
    ar = ai + jnp.int32(0x7FFF) + ((ai >> 16) & 1)
    br = bi + jnp.int32(0x7FFF) + ((bi >> 16) & 1)
    return (ar & jnp.int32(-65536)) | lax.shift_right_logical(br, 16)


def _unpack_bf16_pair(p_i32):
    hi = lax.bitcast_convert_type(p_i32 & jnp.int32(-65536), jnp.float32)
    lo = lax.bitcast_convert_type(p_i32 << 16, jnp.float32)
    return hi, lo


def _router_body(x_ref, gw_ref, logits_ref, p2_ref, wn_ref, be_ref, x32_ref,
                 *, nblk):
    t, d = x_ref.shape
    e = gw_ref.shape[0]
    x = x_ref[...]
    x32_ref[...] = _pack_bf16_pair(x[:, :d // 2], x[:, d // 2:])
    logits = lax.dot_general(x, gw_ref[...], (((1,), (1,)), ((), ())),
                             preferred_element_type=jnp.float32)
    logits_ref[...] = logits
    m = jnp.max(logits, axis=1, keepdims=True)
    p = jnp.exp(logits - m)
    p = p / jnp.sum(p, axis=1, keepdims=True)
    lane = lax.broadcasted_iota(jnp.int32, p.shape, 1)
    work = p
    ohs, mxs = [], []
    for _ in range(TOPK):
        mx = jnp.max(work, axis=1, keepdims=True)
        eq = work == mx
        first = jnp.min(jnp.where(eq, lane, e), axis=1, keepdims=True)
        oh = lane == first
        ohs.append(oh)
        mxs.append(mx)
        work = jnp.where(oh, -1.0, work)
    mask = sum(jnp.where(oh, 1.0, 0.0) for oh in ohs)
    wsum = sum(mxs)
    ri = lax.broadcasted_iota(jnp.int32, (t, t), 0)
    ci = lax.broadcasted_iota(jnp.int32, (t, t), 1)
    tril = jnp.where(ri > ci, 1.0, 0.0)
    rank = jnp.dot(tril, mask, preferred_element_type=jnp.float32)
    counts_row = jnp.dot(jnp.ones((1, t), jnp.float32), mask,
                         preferred_element_type=jnp.float32)
    cpad_row = jnp.floor((counts_row + (BT - 1)) * (1.0 / BT)) * BT
    ui = lax.broadcasted_iota(jnp.int32, (e, e), 0)
    uj = lax.broadcasted_iota(jnp.int32, (e, e), 1)
    strict_u = jnp.where(ui < uj, 1.0, 0.0)
    offs_row = jnp.dot(cpad_row, strict_u,
                       preferred_element_type=jnp.float32)
    base = offs_row + rank
    p2_cols, wn_cols = [], []
    for k in range(TOPK):
        p2_cols.append(jnp.sum(jnp.where(ohs[k], base, 0.0), axis=1,
                               keepdims=True))
        wn_cols.append(mxs[k] / wsum)
    p2_ref[...] = jnp.concatenate(p2_cols, axis=1).astype(jnp.int32)
    wn_ref[...] = jnp.concatenate(wn_cols, axis=1)
    counts_col = lax.dot_general(mask, jnp.ones((t, 1), jnp.float32),
                                 (((0,), (0,)), ((), ())))
    cpad_col = jnp.floor((counts_col + (BT - 1)) * (1.0 / BT)) * BT
    li = lax.broadcasted_iota(jnp.int32, (e, e), 0)
    lj = lax.broadcasted_iota(jnp.int32, (e, e), 1)
    strict_l = jnp.where(li > lj, 1.0, 0.0)
    offs_col = jnp.dot(strict_l, cpad_col,
                       preferred_element_type=jnp.float32)
    bi = lax.broadcasted_iota(jnp.int32, (e, nblk), 1).astype(jnp.float32)
    ge = jnp.where(offs_col <= bi * BT, 1.0, 0.0)
    be_row = jnp.dot(jnp.ones((1, e), jnp.float32), ge,
                     preferred_element_type=jnp.float32) - 1.0
    be_ref[...] = be_row.astype(jnp.int32)


def _run_router(x, gate_w, nblk):
    t, d = x.shape
    e = gate_w.shape[0]
    return pl.pallas_call(
        functools.partial(_router_body, nblk=nblk),
        out_shape=(
            jax.ShapeDtypeStruct((t, e), jnp.float32),
            jax.ShapeDtypeStruct((t, TOPK), jnp.int32),
            jax.ShapeDtypeStruct((t, TOPK), jnp.float32),
            jax.ShapeDtypeStruct((1, nblk), jnp.int32),
            jax.ShapeDtypeStruct((t, d // 2), jnp.int32),
        ),
    )(x, gate_w)


def _sc_scatter_rows(x, p2m, tik, ppad):
    nr, rw = p2m.shape
    d = x.shape[1]
    rpw = nr // 32
    mesh = plsc.VectorSubcoreMesh(core_axis_name="c", subcore_axis_name="s")

    @functools.partial(
        pl.kernel,
        out_type=jax.ShapeDtypeStruct((ppad, d), x.dtype),
        mesh=mesh,
        scratch_types=[
            pltpu.VMEM((rpw, rw), jnp.int32),
            pltpu.VMEM((rpw, rw), jnp.int32),
            pltpu.VMEM((rw, d), x.dtype),
            pltpu.VMEM((rw, d), x.dtype),
            pltpu.SemaphoreType.DMA,
            pltpu.SemaphoreType.DMA,
            pltpu.SemaphoreType.DMA,
            pltpu.SemaphoreType.DMA,
        ],
    )
    def k(x_hbm, p2_hbm, tik_hbm, xs_hbm, idx_v, tik_v, rows0, rows1, gs0,
          gs1, ss0, ss1):
        w = lax.axis_index("s") * 2 + lax.axis_index("c")
        pltpu.sync_copy(p2_hbm.at[pl.ds(w * rpw, rpw)], idx_v)
        pltpu.sync_copy(tik_hbm.at[pl.ds(w * rpw, rpw)], tik_v)
        bufs = (rows0, rows1)
        gsems = (gs0, gs1)
        ssems = (ss0, ss1)
        gc = [None] * rpw
        sc = [None] * rpw
        gc[0] = pltpu.async_copy(x_hbm.at[tik_v.at[0]], bufs[0], gsems[0])
        for r in range(rpw):
            b = r % 2
            if r + 1 < rpw:
                if r >= 1:
                    sc[r - 1].wait()
                gc[r + 1] = pltpu.async_copy(x_hbm.at[tik_v.at[r + 1]],
                                             bufs[1 - b], gsems[1 - b])
            gc[r].wait()
            sc[r] = pltpu.async_copy(bufs[b], xs_hbm.at[idx_v.at[r]],
                                     ssems[b])
        if rpw >= 2:
            sc[rpw - 2].wait()
        sc[rpw - 1].wait()

    return k(x, p2m, tik)


def _sc_gather_back(ys_l, ys_r, p2m):
    nr, rw = p2m.shape
    dh = ys_l.shape[1]
    dt = ys_l.dtype
    rpw = nr // 32
    mesh = plsc.VectorSubcoreMesh(core_axis_name="c", subcore_axis_name="s")

    @functools.partial(
        pl.kernel,
        out_type=(
            jax.ShapeDtypeStruct((nr * rw, dh), dt),
            jax.ShapeDtypeStruct((nr * rw, dh), dt),
        ),
        mesh=mesh,
        scratch_types=[
            pltpu.VMEM((rpw, rw), jnp.int32),
            pltpu.VMEM((rw, dh), dt),
            pltpu.VMEM((rw, dh), dt),
            pltpu.VMEM((rw, dh), dt),
            pltpu.VMEM((rw, dh), dt),
            pltpu.SemaphoreType.DMA,
            pltpu.SemaphoreType.DMA,
            pltpu.SemaphoreType.DMA,
            pltpu.SemaphoreType.DMA,
            pltpu.SemaphoreType.DMA,
            pltpu.SemaphoreType.DMA,
            pltpu.SemaphoreType.DMA,
            pltpu.SemaphoreType.DMA,
        ],
    )
    def k(ysl_hbm, ysr_hbm, p2_hbm, ygl_hbm, ygr_hbm, idx_v, rl0, rl1, rr0,
          rr1, gl0, gl1, gr0, gr1, wl0, wl1, wr0, wr1):
        w = lax.axis_index("s") * 2 + lax.axis_index("c")
        pltpu.sync_copy(p2_hbm.at[pl.ds(w * rpw, rpw)], idx_v)
        lbuf = (rl0, rl1)
        rbuf = (rr0, rr1)
        glsem = (gl0, gl1)
        grsem = (gr0, gr1)
        wlsem = (wl0, wl1)
        wrsem = (wr0, wr1)
        glc = [None] * rpw
        grc = [None] * rpw
        wlc = [None] * rpw
        wrc = [None] * rpw
        glc[0] = pltpu.async_copy(ysl_hbm.at[idx_v.at[0]], lbuf[0], glsem[0])
        grc[0] = pltpu.async_copy(ysr_hbm.at[idx_v.at[0]], rbuf[0], grsem[0])
        for r in range(rpw):
            b = r % 2
            if r + 1 < rpw:
                if r >= 1:
                    wlc[r - 1].wait()
                    wrc[r - 1].wait()
                glc[r + 1] = pltpu.async_copy(ysl_hbm.at[idx_v.at[r + 1]],
                                              lbuf[1 - b], glsem[1 - b])
                grc[r + 1] = pltpu.async_copy(ysr_hbm.at[idx_v.at[r + 1]],
                                              rbuf[1 - b], grsem[1 - b])
            glc[r].wait()
            grc[r].wait()
            dst = pl.ds((w * rpw + r) * rw, rw)
            wlc[r] = pltpu.async_copy(lbuf[b], ygl_hbm.at[dst], wlsem[b])
            wrc[r] = pltpu.async_copy(rbuf[b], ygr_hbm.at[dst], wrsem[b])
        if rpw >= 2:
            wlc[rpw - 2].wait()
            wrc[rpw - 2].wait()
        wlc[rpw - 1].wait()
        wrc[rpw - 1].wait()

    return k(ys_l, ys_r, p2m)


def _ffn_body(be_ref, xs_ref, wg_ref, wu_ref, wd_ref, ysl_ref, ysr_ref):
    xhi, xlo = _unpack_bf16_pair(xs_ref[...])
    xb = jnp.concatenate([xhi, xlo], axis=1).astype(jnp.bfloat16)
    h = jnp.dot(xb, wg_ref[0], preferred_element_type=jnp.float32)
    u = jnp.dot(xb, wu_ref[0], preferred_element_type=jnp.float32)
    act = h * jax.nn.sigmoid(h) * u
    y = jnp.dot(act.astype(jnp.bfloat16), wd_ref[0],
                preferred_element_type=jnp.float32)
    d = y.shape[1]
    dh = d // 2
    dq = d // 4
    yl = y[:, :dh]
    yr = y[:, dh:]
    ysl_ref[...] = _pack_bf16_pair(yl[:, :dq], yl[:, dq:])
    ysr_ref[...] = _pack_bf16_pair(yr[:, :dq], yr[:, dq:])


def _run_ffn(be1d, xs32, wg, wu, wd):
    nblk = be1d.shape[0]
    _, d, f = wg.shape
    ppad = xs32.shape[0]
    dq = d // 4
    grid_spec = pltpu.PrefetchScalarGridSpec(
        num_scalar_prefetch=1,
        grid=(nblk,),
        in_specs=[
            pl.BlockSpec((BT, d // 2), lambda i, be: (i, 0)),
            pl.BlockSpec((1, d, f), lambda i, be: (be[i], 0, 0)),
            pl.BlockSpec((1, d, f), lambda i, be: (be[i], 0, 0)),
            pl.BlockSpec((1, f, d), lambda i, be: (be[i], 0, 0)),
        ],
        out_specs=(
            pl.BlockSpec((BT, dq), lambda i, be: (i, 0)),
            pl.BlockSpec((BT, dq), lambda i, be: (i, 0)),
        ),
    )
    return pl.pallas_call(
        _ffn_body,
        grid_spec=grid_spec,
        out_shape=(
            jax.ShapeDtypeStruct((ppad, dq), jnp.int32),
            jax.ShapeDtypeStruct((ppad, dq), jnp.int32),
        ),
    )(be1d, xs32, wg, wu, wd)


def _shared_body(x_ref, shg_ref, shu_ref, shd_ref, segw_ref, out_ref, *,
                 nchunk):
    j = pl.program_id(1)
    x = x_ref[...]
    xb = x.astype(jnp.bfloat16)
    g = jnp.dot(xb, shg_ref[...], preferred_element_type=jnp.float32)
    u = jnp.dot(xb, shu_ref[...], preferred_element_type=jnp.float32)
    s = jnp.dot((g * jax.nn.sigmoid(g) * u).astype(jnp.bfloat16),
                shd_ref[...], preferred_element_type=jnp.float32)

    @pl.when(j == 0)
    def _():
        out_ref[...] = s

    @pl.when(j != 0)
    def _():
        out_ref[...] += s

    @pl.when(j == nchunk - 1)
    def _():
        gate = jax.nn.sigmoid(jnp.dot(x, segw_ref[...],
                                      preferred_element_type=jnp.float32))
        out_ref[...] = gate * out_ref[...]


def _run_shared(x, sh_gate_w, sh_up_w, sh_down_w, seg_w):
    t, d = x.shape
    sf = sh_gate_w.shape[1]
    nchunk = 1
    cf = sf // nchunk
    bt = 512 if t % 512 == 0 else t
    return pl.pallas_call(
        functools.partial(_shared_body, nchunk=nchunk),
        grid=(t // bt, nchunk),
        in_specs=[
            pl.BlockSpec((bt, d), lambda i, j: (i, 0)),
            pl.BlockSpec((d, cf), lambda i, j: (0, j)),
            pl.BlockSpec((d, cf), lambda i, j: (0, j)),
            pl.BlockSpec((cf, d), lambda i, j: (j, 0)),
            pl.BlockSpec((d, 1), lambda i, j: (0, 0)),
        ],
        out_specs=pl.BlockSpec((bt, d), lambda i, j: (i, 0)),
        out_shape=jax.ShapeDtypeStruct((t, d), jnp.float32),
    )(x, sh_gate_w, sh_up_w, sh_down_w, seg_w)


def _combine_body(ygl_ref, ygr_ref, wn_ref, sh_ref, out_ref):
    dq = ygl_ref.shape[1] // TOPK
    dh = 2 * dq
    wn = wn_ref[...]
    acc = [sh_ref[:, i * dq:(i + 1) * dq] for i in range(4)]
    for k in range(TOPK):
        wk = wn[:, k:k + 1]
        lhi, llo = _unpack_bf16_pair(ygl_ref[:, k * dq:(k + 1) * dq])
        rhi, rlo = _unpack_bf16_pair(ygr_ref[:, k * dq:(k + 1) * dq])
        acc[0] = acc[0] + wk * lhi
        acc[1] = acc[1] + wk * llo
        acc[2] = acc[2] + wk * rhi
        acc[3] = acc[3] + wk * rlo
    for i in range(4):
        out_ref[:, i * dq:(i + 1) * dq] = acc[i]


def _run_combine(yg_l2, yg_r2, wn, shared):
    t, d = shared.shape
    btok = 256 if t % 256 == 0 else t
    grid = (t // btok,)
    kd = yg_l2.shape[1]
    return pl.pallas_call(
        _combine_body,
        grid=grid,
        in_specs=[
            pl.BlockSpec((btok, kd), lambda i: (i, 0)),
            pl.BlockSpec((btok, kd), lambda i: (i, 0)),
            pl.BlockSpec((btok, TOPK), lambda i: (i, 0)),
            pl.BlockSpec((btok, d), lambda i: (i, 0)),
        ],
        out_specs=pl.BlockSpec((btok, d), lambda i: (i, 0)),
        out_shape=jax.ShapeDtypeStruct((t, d), jnp.float32),
    )(yg_l2, yg_r2, wn, shared)


def kernel(hidden_states, gate_w, W_gate, W_up, W_down, sh_gate_w, sh_up_w,
           sh_down_w, shared_expert_gate_w):
    b, s, d = hidden_states.shape
    t = b * s
    e, _, f = W_gate.shape
    x = hidden_states.reshape(t, d)
    nblk = (t * TOPK) // BT + e
    ppad = nblk * BT

    logits, p2, wn, be2, x32 = _run_router(x, gate_w, nblk)
    p2m = p2.reshape(-1, 64)
    tik = (jnp.arange(t * TOPK, dtype=jnp.int32) // TOPK).reshape(-1, 64)

    xs32 = _sc_scatter_rows(x32, p2m, tik, ppad)

    ys_l, ys_r = _run_ffn(be2.reshape(nblk), xs32,
                          W_gate.astype(jnp.bfloat16),
                          W_up.astype(jnp.bfloat16),
                          W_down.astype(jnp.bfloat16))

    shared = _run_shared(x, sh_gate_w.astype(jnp.bfloat16),
                         sh_up_w.astype(jnp.bfloat16),
                         sh_down_w.astype(jnp.bfloat16),
                         shared_expert_gate_w)

    dq = d // 4
    yg_l, yg_r = _sc_gather_back(ys_l, ys_r, p2m)

    out = _run_combine(yg_l.reshape(t, TOPK * dq),
                       yg_r.reshape(t, TOPK * dq), wn, shared)
    return (out.reshape(b, s, d), logits)


# speedup vs baseline: 15.4525x; 1.0017x over previous
import functools

import jax
import jax.numpy as jnp
from jax import lax
from jax.experimental import pallas as pl
from jax.experimental.pallas import tpu as pltpu
from jax.experimental.pallas import tpu_sc as plsc

TOPK = 8
BT = 256


def _pack_bf16_pair(a_f32, b_f32):
    ai = lax.bitcast_convert_type(a_f32, jnp.int32)
    bi = lax.bitcast_convert_type(b_f32, jnp.int32)
    ar = ai + jnp.int32(0x7FFF) + ((ai >> 16) & 1)
    br = bi + jnp.int32(0x7FFF) + ((bi >> 16) & 1)
    return (ar & jnp.int32(-65536)) | lax.shift_right_logical(br, 16)


def _unpack_bf16_pair(p_i32):
    hi = lax.bitcast_convert_type(p_i32 & jnp.int32(-65536), jnp.float32)
    lo = lax.bitcast_convert_type(p_i32 << 16, jnp.float32)
    return hi, lo


def _router_body(x_ref, gw_ref, logits_ref, p2_ref, wn_ref, be_ref, x32_ref,
                 *, nblk):
    t, d = x_ref.shape
    e = gw_ref.shape[0]
    x = x_ref[...]
    x32_ref[...] = _pack_bf16_pair(x[:, :d // 2], x[:, d // 2:])
    logits = lax.dot_general(x, gw_ref[...], (((1,), (1,)), ((), ())),
                             preferred_element_type=jnp.float32)
    logits_ref[...] = logits
    m = jnp.max(logits, axis=1, keepdims=True)
    p = jnp.exp(logits - m)
    p = p / jnp.sum(p, axis=1, keepdims=True)
    lane = lax.broadcasted_iota(jnp.int32, p.shape, 1)
    work = p
    ohs, mxs = [], []
    for _ in range(TOPK):
        mx = jnp.max(work, axis=1, keepdims=True)
        eq = work == mx
        first = jnp.min(jnp.where(eq, lane, e), axis=1, keepdims=True)
        oh = lane == first
        ohs.append(oh)
        mxs.append(mx)
        work = jnp.where(oh, -1.0, work)
    mask = sum(jnp.where(oh, 1.0, 0.0) for oh in ohs)
    wsum = sum(mxs)
    ri = lax.broadcasted_iota(jnp.int32, (t, t), 0)
    ci = lax.broadcasted_iota(jnp.int32, (t, t), 1)
    tril = jnp.where(ri > ci, 1.0, 0.0)
    rank = jnp.dot(tril, mask, preferred_element_type=jnp.float32)
    counts_row = jnp.dot(jnp.ones((1, t), jnp.float32), mask,
                         preferred_element_type=jnp.float32)
    cpad_row = jnp.floor((counts_row + (BT - 1)) * (1.0 / BT)) * BT
    ui = lax.broadcasted_iota(jnp.int32, (e, e), 0)
    uj = lax.broadcasted_iota(jnp.int32, (e, e), 1)
    strict_u = jnp.where(ui < uj, 1.0, 0.0)
    offs_row = jnp.dot(cpad_row, strict_u,
                       preferred_element_type=jnp.float32)
    base = offs_row + rank
    p2_cols, wn_cols = [], []
    for k in range(TOPK):
        p2_cols.append(jnp.sum(jnp.where(ohs[k], base, 0.0), axis=1,
                               keepdims=True))
        wn_cols.append(mxs[k] / wsum)
    p2_ref[...] = jnp.concatenate(p2_cols, axis=1).astype(jnp.int32)
    wn_ref[...] = jnp.concatenate(wn_cols, axis=1)
    counts_col = lax.dot_general(mask, jnp.ones((t, 1), jnp.float32),
                                 (((0,), (0,)), ((), ())))
    cpad_col = jnp.floor((counts_col + (BT - 1)) * (1.0 / BT)) * BT
    li = lax.broadcasted_iota(jnp.int32, (e, e), 0)
    lj = lax.broadcasted_iota(jnp.int32, (e, e), 1)
    strict_l = jnp.where(li > lj, 1.0, 0.0)
    offs_col = jnp.dot(strict_l, cpad_col,
                       preferred_element_type=jnp.float32)
    bi = lax.broadcasted_iota(jnp.int32, (e, nblk), 1).astype(jnp.float32)
    ge = jnp.where(offs_col <= bi * BT, 1.0, 0.0)
    be_row = jnp.dot(jnp.ones((1, e), jnp.float32), ge,
                     preferred_element_type=jnp.float32) - 1.0
    be_ref[...] = be_row.astype(jnp.int32)


def _run_router(x, gate_w, nblk):
    t, d = x.shape
    e = gate_w.shape[0]
    return pl.pallas_call(
        functools.partial(_router_body, nblk=nblk),
        out_shape=(
            jax.ShapeDtypeStruct((t, e), jnp.float32),
            jax.ShapeDtypeStruct((t, TOPK), jnp.int32),
            jax.ShapeDtypeStruct((t, TOPK), jnp.float32),
            jax.ShapeDtypeStruct((1, nblk), jnp.int32),
            jax.ShapeDtypeStruct((t, d // 2), jnp.int32),
        ),
    )(x, gate_w)


def _sc_scatter_rows(x, p2m, tik, ppad):
    nr, rw = p2m.shape
    d = x.shape[1]
    rpw = nr // 32
    mesh = plsc.VectorSubcoreMesh(core_axis_name="c", subcore_axis_name="s")

    @functools.partial(
        pl.kernel,
        out_type=jax.ShapeDtypeStruct((ppad, d), x.dtype),
        mesh=mesh,
        scratch_types=[
            pltpu.VMEM((rpw, rw), jnp.int32),
            pltpu.VMEM((rpw, rw), jnp.int32),
            pltpu.VMEM((rw, d), x.dtype),
            pltpu.VMEM((rw, d), x.dtype),
            pltpu.SemaphoreType.DMA,
            pltpu.SemaphoreType.DMA,
            pltpu.SemaphoreType.DMA,
            pltpu.SemaphoreType.DMA,
        ],
    )
    def k(x_hbm, p2_hbm, tik_hbm, xs_hbm, idx_v, tik_v, rows0, rows1, gs0,
          gs1, ss0, ss1):
        w = lax.axis_index("s") * 2 + lax.axis_index("c")
        pltpu.sync_copy(p2_hbm.at[pl.ds(w * rpw, rpw)], idx_v)
        pltpu.sync_copy(tik_hbm.at[pl.ds(w * rpw, rpw)], tik_v)
        bufs = (rows0, rows1)
        gsems = (gs0, gs1)
        ssems = (ss0, ss1)
        gc = [None] * rpw
        sc = [None] * rpw
        gc[0] = pltpu.async_copy(x_hbm.at[tik_v.at[0]], bufs[0], gsems[0])
        for r in range(rpw):
            b = r % 2
            if r + 1 < rpw:
                if r >= 1:
                    sc[r - 1].wait()
                gc[r + 1] = pltpu.async_copy(x_hbm.at[tik_v.at[r + 1]],
                                             bufs[1 - b], gsems[1 - b])
            gc[r].wait()
            sc[r] = pltpu.async_copy(bufs[b], xs_hbm.at[idx_v.at[r]],
                                     ssems[b])
        if rpw >= 2:
            sc[rpw - 2].wait()
        sc[rpw - 1].wait()

    return k(x, p2m, tik)


def _sc_gather_back(ys_l, ys_r, p2m):
    nr, rw = p2m.shape
    dh = ys_l.shape[1]
    dt = ys_l.dtype
    rpw = nr // 32
    mesh = plsc.VectorSubcoreMesh(core_axis_name="c", subcore_axis_name="s")

    @functools.partial(
        pl.kernel,
        out_type=(
            jax.ShapeDtypeStruct((nr * rw, dh), dt),
            jax.ShapeDtypeStruct((nr * rw, dh), dt),
        ),
        mesh=mesh,
        scratch_types=[
            pltpu.VMEM((rpw, rw), jnp.int32),
            pltpu.VMEM((rw, dh), dt),
            pltpu.VMEM((rw, dh), dt),
            pltpu.VMEM((rw, dh), dt),
            pltpu.VMEM((rw, dh), dt),
            pltpu.SemaphoreType.DMA,
            pltpu.SemaphoreType.DMA,
            pltpu.SemaphoreType.DMA,
            pltpu.SemaphoreType.DMA,
            pltpu.SemaphoreType.DMA,
            pltpu.SemaphoreType.DMA,
            pltpu.SemaphoreType.DMA,
            pltpu.SemaphoreType.DMA,
        ],
    )
    def k(ysl_hbm, ysr_hbm, p2_hbm, ygl_hbm, ygr_hbm, idx_v, rl0, rl1, rr0,
          rr1, gl0, gl1, gr0, gr1, wl0, wl1, wr0, wr1):
        w = lax.axis_index("s") * 2 + lax.axis_index("c")
        pltpu.sync_copy(p2_hbm.at[pl.ds(w * rpw, rpw)], idx_v)
        lbuf = (rl0, rl1)
        rbuf = (rr0, rr1)
        glsem = (gl0, gl1)
        grsem = (gr0, gr1)
        wlsem = (wl0, wl1)
        wrsem = (wr0, wr1)
        glc = [None] * rpw
        grc = [None] * rpw
        wlc = [None] * rpw
        wrc = [None] * rpw
        glc[0] = pltpu.async_copy(ysl_hbm.at[idx_v.at[0]], lbuf[0], glsem[0])
        grc[0] = pltpu.async_copy(ysr_hbm.at[idx_v.at[0]], rbuf[0], grsem[0])
        for r in range(rpw):
            b = r % 2
            if r + 1 < rpw:
                if r >= 1:
                    wlc[r - 1].wait()
                    wrc[r - 1].wait()
                glc[r + 1] = pltpu.async_copy(ysl_hbm.at[idx_v.at[r + 1]],
                                              lbuf[1 - b], glsem[1 - b])
                grc[r + 1] = pltpu.async_copy(ysr_hbm.at[idx_v.at[r + 1]],
                                              rbuf[1 - b], grsem[1 - b])
            glc[r].wait()
            grc[r].wait()
            dst = pl.ds((w * rpw + r) * rw, rw)
            wlc[r] = pltpu.async_copy(lbuf[b], ygl_hbm.at[dst], wlsem[b])
            wrc[r] = pltpu.async_copy(rbuf[b], ygr_hbm.at[dst], wrsem[b])
        if rpw >= 2:
            wlc[rpw - 2].wait()
            wrc[rpw - 2].wait()
        wlc[rpw - 1].wait()
        wrc[rpw - 1].wait()

    return k(ys_l, ys_r, p2m)


def _ffn_body(be_ref, xs_ref, wg_ref, wu_ref, wd_ref, ysl_ref, ysr_ref):
    xhi, xlo = _unpack_bf16_pair(xs_ref[...])
    xb = jnp.concatenate([xhi, xlo], axis=1).astype(jnp.bfloat16)
    h = jnp.dot(xb, wg_ref[0], preferred_element_type=jnp.float32)
    u = jnp.dot(xb, wu_ref[0], preferred_element_type=jnp.float32)
    act = h * jax.nn.sigmoid(h) * u
    y = jnp.dot(act.astype(jnp.bfloat16), wd_ref[0],
                preferred_element_type=jnp.float32)
    d = y.shape[1]
    dh = d // 2
    dq = d // 4
    yl = y[:, :dh]
    yr = y[:, dh:]
    ysl_ref[...] = _pack_bf16_pair(yl[:, :dq], yl[:, dq:])
    ysr_ref[...] = _pack_bf16_pair(yr[:, :dq], yr[:, dq:])


def _run_ffn(be1d, xs32, wg, wu, wd):
    nblk = be1d.shape[0]
    _, d, f = wg.shape
    ppad = xs32.shape[0]
    dq = d // 4
    grid_spec = pltpu.PrefetchScalarGridSpec(
        num_scalar_prefetch=1,
        grid=(nblk,),
        in_specs=[
            pl.BlockSpec((BT, d // 2), lambda i, be: (i, 0)),
            pl.BlockSpec((1, d, f), lambda i, be: (be[i], 0, 0)),
            pl.BlockSpec((1, d, f), lambda i, be: (be[i], 0, 0)),
            pl.BlockSpec((1, f, d), lambda i, be: (be[i], 0, 0)),
        ],
        out_specs=(
            pl.BlockSpec((BT, dq), lambda i, be: (i, 0)),
            pl.BlockSpec((BT, dq), lambda i, be: (i, 0)),
        ),
    )
    return pl.pallas_call(
        _ffn_body,
        grid_spec=grid_spec,
        out_shape=(
            jax.ShapeDtypeStruct((ppad, dq), jnp.int32),
            jax.ShapeDtypeStruct((ppad, dq), jnp.int32),
        ),
    )(be1d, xs32, wg, wu, wd)


def _shared_body(x_ref, shg_ref, shu_ref, shd_ref, segw_ref, out_ref, *,
                 nchunk):
    j = pl.program_id(1)
    x = x_ref[...]
    xb = x.astype(jnp.bfloat16)
    g = jnp.dot(xb, shg_ref[...], preferred_element_type=jnp.float32)
    u = jnp.dot(xb, shu_ref[...], preferred_element_type=jnp.float32)
    s = jnp.dot((g * jax.nn.sigmoid(g) * u).astype(jnp.bfloat16),
                shd_ref[...], preferred_element_type=jnp.float32)

    @pl.when(j == 0)
    def _():
        out_ref[...] = s

    @pl.when(j != 0)
    def _():
        out_ref[...] += s

    @pl.when(j == nchunk - 1)
    def _():
        gate = jax.nn.sigmoid(jnp.dot(x, segw_ref[...],
                                      preferred_element_type=jnp.float32))
        out_ref[...] = gate * out_ref[...]


def _run_shared(x, sh_gate_w, sh_up_w, sh_down_w, seg_w):
    t, d = x.shape
    sf = sh_gate_w.shape[1]
    nchunk = 1
    cf = sf // nchunk
    bt = 512 if t % 512 == 0 else t
    return pl.pallas_call(
        functools.partial(_shared_body, nchunk=nchunk),
        grid=(t // bt, nchunk),
        in_specs=[
            pl.BlockSpec((bt, d), lambda i, j: (i, 0)),
            pl.BlockSpec((d, cf), lambda i, j: (0, j)),
            pl.BlockSpec((d, cf), lambda i, j: (0, j)),
            pl.BlockSpec((cf, d), lambda i, j: (j, 0)),
            pl.BlockSpec((d, 1), lambda i, j: (0, 0)),
        ],
        out_specs=pl.BlockSpec((bt, d), lambda i, j: (i, 0)),
        out_shape=jax.ShapeDtypeStruct((t, d), jnp.float32),
    )(x, sh_gate_w, sh_up_w, sh_down_w, seg_w)


def _combine_body(ygl_ref, ygr_ref, wn_ref, sh_ref, out_ref):
    dq = ygl_ref.shape[1] // TOPK
    dh = 2 * dq
    wn = wn_ref[...]
    acc = [sh_ref[:, i * dq:(i + 1) * dq] for i in range(4)]
    for k in range(TOPK):
        wk = wn[:, k:k + 1]
        lhi, llo = _unpack_bf16_pair(ygl_ref[:, k * dq:(k + 1) * dq])
        rhi, rlo = _unpack_bf16_pair(ygr_ref[:, k * dq:(k + 1) * dq])
        acc[0] = acc[0] + wk * lhi
        acc[1] = acc[1] + wk * llo
        acc[2] = acc[2] + wk * rhi
        acc[3] = acc[3] + wk * rlo
    for i in range(4):
        out_ref[:, i * dq:(i + 1) * dq] = acc[i]


def _run_combine(yg_l2, yg_r2, wn, shared):
    t, d = shared.shape
    btok = 256 if t % 256 == 0 else t
    grid = (t // btok,)
    kd = yg_l2.shape[1]
    return pl.pallas_call(
        _combine_body,
        grid=grid,
        in_specs=[
            pl.BlockSpec((btok, kd), lambda i: (i, 0)),
            pl.BlockSpec((btok, kd), lambda i: (i, 0)),
            pl.BlockSpec((btok, TOPK), lambda i: (i, 0)),
            pl.BlockSpec((btok, d), lambda i: (i, 0)),
        ],
        out_specs=pl.BlockSpec((btok, d), lambda i: (i, 0)),
        out_shape=jax.ShapeDtypeStruct((t, d), jnp.float32),
    )(yg_l2, yg_r2, wn, shared)


def kernel(hidden_states, gate_w, W_gate, W_up, W_down, sh_gate_w, sh_up_w,
           sh_down_w, shared_expert_gate_w):
    b, s, d = hidden_states.shape
    t = b * s
    e, _, f = W_gate.shape
    x = hidden_states.reshape(t, d)
    nblk = (t * TOPK) // BT + e
    ppad = nblk * BT

    logits, p2, wn, be2, x32 = _run_router(x, gate_w, nblk)
    p2m = p2.reshape(-1, 64)
    tik = (jnp.arange(t * TOPK, dtype=jnp.int32) // TOPK).reshape(-1, 64)

    shared = _run_shared(x, sh_gate_w.astype(jnp.bfloat16),
                         sh_up_w.astype(jnp.bfloat16),
                         sh_down_w.astype(jnp.bfloat16),
                         shared_expert_gate_w)

    xs32 = _sc_scatter_rows(x32, p2m, tik, ppad)

    ys_l, ys_r = _run_ffn(be2.reshape(nblk), xs32,
                          W_gate.astype(jnp.bfloat16),
                          W_up.astype(jnp.bfloat16),
                          W_down.astype(jnp.bfloat16))

    dq = d // 4
    yg_l, yg_r = _sc_gather_back(ys_l, ys_r, p2m)

    out = _run_combine(yg_l.reshape(t, TOPK * dq),
                       yg_r.reshape(t, TOPK * dq), wn, shared)
    return (out.reshape(b, s, d), logits)
